# baseline (speedup 1.0000x reference)
import functools

import jax
import jax.numpy as jnp
from jax import lax
from jax.experimental import pallas as pl
from jax.experimental.pallas import tpu as pltpu
from jax.experimental.pallas import tpu_sc as plsc

f32 = jnp.float32
i32 = jnp.int32

N = 100000
E = 1600000
EP = 1638400
M = EP // 128

_mesh = plsc.VectorSubcoreMesh(core_axis_name="c", subcore_axis_name="s")
_params = pltpu.CompilerParams(needs_layout_passes=False,
                               use_tc_tiling_on_sc=False)


def _nr_rsqrt(x):
    i = lax.bitcast_convert_type(x, i32)
    i = jnp.int32(0x5F3759DF) - lax.shift_right_arithmetic(
        i, jnp.full((16,), 1, i32))
    y = lax.bitcast_convert_type(i, f32)
    for _ in range(3):
        y = y * (1.5 - 0.5 * x * y * y)
    return jnp.where(x > 0.0, y, jnp.zeros_like(y))


@functools.partial(
    pl.kernel,
    compiler_params=_params,
    out_type=jax.ShapeDtypeStruct((M, 128), f32),
    mesh=_mesh,
    scratch_types=[
        pltpu.VMEM((2, 8, 2, 128), i32),
        pltpu.VMEM((2, 8, 128), f32),
        pltpu.VMEM((2, 8, 128), f32),
        pltpu.VMEM((N,), f32),
        pltpu.VMEM((6256,), f32),
        pltpu.VMEM_SHARED((N,), f32),
        pltpu.SemaphoreType.DMA,
        pltpu.SemaphoreType.DMA,
        pltpu.SemaphoreType.DMA,
        pltpu.SemaphoreType.DMA,
    ],
)
def _norm_kernel(ei_h, ew_h, out_h, eibuf, ebuf, mbuf, dx, tbuf, dacc,
                 sl0, sl1, sx0, sx1):
    c = lax.axis_index("c")
    s = lax.axis_index("s")
    zoff = jnp.minimum(s * 6256, N - 6256)
    sl = (sl0, sl1)
    sx = (sx0, sx1)

    def fire_lin(p, b):
        pltpu.async_copy(ei_h.at[pl.ds(b, 8)], eibuf.at[p], sl[p])
        pltpu.async_copy(ew_h.at[pl.ds(b, 8)], ebuf.at[p], sl[p])

    def wait_lin(p):
        pltpu.make_async_copy(ei_h.at[pl.ds(0, 8)], eibuf.at[p], sl[p]).wait()
        pltpu.make_async_copy(ew_h.at[pl.ds(0, 8)], ebuf.at[p], sl[p]).wait()

    def _zb(i, carry):
        tbuf[pl.ds(i * 16, 16)] = jnp.zeros((16,), f32)
        return carry

    lax.fori_loop(0, 6256 // 16, _zb, 0)
    pltpu.sync_copy(tbuf, dacc.at[pl.ds(zoff, 6256)])
    plsc.subcore_barrier()

    ra0 = s * 800
    ramax = ra0 + 800 - 8

    def comp_a(p):
        for j in range(8):
            @plsc.parallel_loop(0, 128, 16, unroll=2)
            def _mk(e):
                sv = eibuf[p, j, 0, pl.ds(e, 16)]
                dv = eibuf[p, j, 1, pl.ds(e, 16)]
                ev = ebuf[p, j, pl.ds(e, 16)]
                mbuf[p, j, pl.ds(e, 16)] = jnp.where(sv == dv, 0.0, ev)

    def fire_sct(p):
        for j in range(8):
            pltpu.async_copy(mbuf.at[p, j], dacc.at[eibuf.at[p, j, 0]],
                             sx[p], add=True)

    def wait_sct(p):
        for j in range(8):
            pltpu.make_async_copy(mbuf.at[p, j], dacc.at[eibuf.at[p, j, 0]],
                                  sx[p]).wait()

    fire_lin(0, ra0)
    fire_lin(1, ra0 + 8)

    def body_a(gg, carry):
        g0 = 2 * gg
        wait_lin(0)
        comp_a(0)
        fire_sct(0)
        wait_lin(1)
        comp_a(1)
        fire_sct(1)
        wait_sct(0)
        fire_lin(0, jnp.minimum(ra0 + (g0 + 2) * 8, ramax))
        wait_sct(1)
        fire_lin(1, jnp.minimum(ra0 + (g0 + 3) * 8, ramax))
        return carry

    lax.fori_loop(0, 50, body_a, 0)
    wait_lin(0)
    wait_lin(1)
    plsc.subcore_barrier()

    pltpu.sync_copy(dacc.at[pl.ds(zoff, 6256)], tbuf)
    plsc.subcore_barrier()

    @plsc.parallel_loop(0, 6256, 16, unroll=2)
    def _rs(i):
        tbuf[pl.ds(i, 16)] = _nr_rsqrt(tbuf[pl.ds(i, 16)])

    pltpu.sync_copy(tbuf, dacc.at[pl.ds(zoff, 6256)])
    plsc.subcore_barrier()
    pltpu.sync_copy(dacc, dx)

    rc0 = c * 6400 + s * 400

    def comp_c(p):
        for j in range(8):
            @plsc.parallel_loop(0, 128, 16, unroll=2)
            def _mk(e):
                sv = eibuf[p, j, 0, pl.ds(e, 16)]
                dv = eibuf[p, j, 1, pl.ds(e, 16)]
                ev = ebuf[p, j, pl.ds(e, 16)]
                a = plsc.load_gather(dx, [sv])
                b = plsc.load_gather(dx, [dv])
                ewp = jnp.where(sv == dv, 0.0, ev)
                mbuf[p, j, pl.ds(e, 16)] = (-a) * ewp * b

    def fire_out(p, b):
        pltpu.async_copy(mbuf.at[p], out_h.at[pl.ds(b, 8)], sx[p])

    def wait_out(p):
        pltpu.make_async_copy(mbuf.at[p], out_h.at[pl.ds(0, 8)], sx[p]).wait()

    rcmax = rc0 + 400 - 8
    fire_lin(0, rc0)
    fire_lin(1, rc0 + 8)

    def body_c(gg, carry):
        g0 = 2 * gg
        wait_lin(0)
        comp_c(0)
        fire_out(0, rc0 + g0 * 8)
        wait_lin(1)
        comp_c(1)
        fire_out(1, rc0 + (g0 + 1) * 8)
        wait_out(0)
        fire_lin(0, jnp.minimum(rc0 + (g0 + 2) * 8, rcmax))
        wait_out(1)
        fire_lin(1, jnp.minimum(rc0 + (g0 + 3) * 8, rcmax))
        return carry

    lax.fori_loop(0, 25, body_c, 0)
    wait_lin(0)
    wait_lin(1)


@functools.partial(
    pl.kernel,
    compiler_params=_params,
    out_type=jax.ShapeDtypeStruct((N,), f32),
    mesh=_mesh,
    scratch_types=[
        pltpu.VMEM((2, 8, 2, 128), i32),
        pltpu.VMEM((2, 8, 128), f32),
        pltpu.VMEM((2, 8, 128), f32),
        pltpu.VMEM((N,), f32),
        pltpu.VMEM_SHARED((N,), f32),
        pltpu.SemaphoreType.DMA,
        pltpu.SemaphoreType.DMA,
        pltpu.SemaphoreType.DMA,
        pltpu.SemaphoreType.DMA,
    ],
)
def _prop1_kernel(ei_h, nrm_h, x_h, z1_h, out_h, eibuf, nbuf, mbuf, xbuf, acc,
                  sl0, sl1, sx0, sx1):
    c = lax.axis_index("c")
    s = lax.axis_index("s")
    zoff = jnp.minimum(s * 6256, N - 6256)
    sl = (sl0, sl1)
    sx = (sx0, sx1)
    pltpu.sync_copy(z1_h, acc.at[pl.ds(zoff, 6256)])
    pltpu.sync_copy(x_h, xbuf)
    plsc.subcore_barrier()

    r0 = s * 800
    rmax = r0 + 800 - 8

    def fire_lin(p, b):
        pltpu.async_copy(ei_h.at[pl.ds(b, 8)], eibuf.at[p], sl[p])
        pltpu.async_copy(nrm_h.at[pl.ds(b, 8)], nbuf.at[p], sl[p])

    def wait_lin(p):
        pltpu.make_async_copy(ei_h.at[pl.ds(0, 8)], eibuf.at[p], sl[p]).wait()
        pltpu.make_async_copy(nrm_h.at[pl.ds(0, 8)], nbuf.at[p], sl[p]).wait()

    def comp(p):
        for j in range(8):
            @plsc.parallel_loop(0, 128, 16, unroll=2)
            def _mk(e):
                sv = eibuf[p, j, 0, pl.ds(e, 16)]
                xg = plsc.load_gather(xbuf, [sv])
                mbuf[p, j, pl.ds(e, 16)] = xg * nbuf[p, j, pl.ds(e, 16)]

    def fire_sct(p):
        for j in range(8):
            pltpu.async_copy(mbuf.at[p, j], acc.at[eibuf.at[p, j, 1]],
                             sx[p], add=True)

    def wait_sct(p):
        for j in range(8):
            pltpu.make_async_copy(mbuf.at[p, j], acc.at[eibuf.at[p, j, 1]],
                                  sx[p]).wait()

    fire_lin(0, r0)
    fire_lin(1, r0 + 8)

    def body(gg, carry):
        g0 = 2 * gg
        wait_lin(0)
        comp(0)
        fire_sct(0)
        wait_lin(1)
        comp(1)
        fire_sct(1)
        wait_sct(0)
        fire_lin(0, jnp.minimum(r0 + (g0 + 2) * 8, rmax))
        wait_sct(1)
        fire_lin(1, jnp.minimum(r0 + (g0 + 3) * 8, rmax))
        return carry

    lax.fori_loop(0, 50, body, 0)
    wait_lin(0)
    wait_lin(1)
    plsc.subcore_barrier()
    off = c * 50000 + jnp.minimum(s * 3128, 50000 - 3128)
    pltpu.sync_copy(acc.at[pl.ds(off, 3128)], xbuf.at[pl.ds(0, 3128)])
    pltpu.sync_copy(xbuf.at[pl.ds(0, 3128)], out_h.at[pl.ds(off, 3128)])


def _prop_wide_body(c, s, ei_h, nrm_h, out_h, z2_h, eibuf, nbuf, rows, acc,
                    sems, r0, rmax, nch, fire_gat, wait_gat):
    sl = (sems[0], sems[1])
    sg = (sems[2], sems[3])
    ss = (sems[4], sems[5])
    zoff = jnp.minimum(s * 6256, N - 6256)
    for k in range(2):
        pltpu.sync_copy(z2_h, acc.at[pl.ds(zoff + k * 3128, 3128)])
    plsc.subcore_barrier()

    def fire_lin(p, b):
        pltpu.async_copy(ei_h.at[pl.ds(b, 4)], eibuf.at[p], sl[p])
        pltpu.async_copy(nrm_h.at[pl.ds(b, 4)], nbuf.at[p], sl[p])

    def wait_lin(p):
        pltpu.make_async_copy(ei_h.at[pl.ds(0, 4)], eibuf.at[p], sl[p]).wait()
        pltpu.make_async_copy(nrm_h.at[pl.ds(0, 4)], nbuf.at[p], sl[p]).wait()

    def scale(p):
        for j in range(4):
            @plsc.parallel_loop(0, 128, 16)
            def _sc(e):
                nv = nbuf[p, j, pl.ds(e, 16)]
                for kk in range(16):
                    idx = j * 128 + e + kk
                    rows[p, idx, :] = rows[p, idx, :] * nv[kk]

    def fire_sct(p):
        for j in range(4):
            pltpu.async_copy(rows.at[p, pl.ds(j * 128, 128)],
                             acc.at[eibuf.at[p, j, 1]], ss[p], add=True)

    def wait_sct(p):
        for j in range(4):
            pltpu.make_async_copy(rows.at[p, pl.ds(j * 128, 128)],
                                  acc.at[eibuf.at[p, j, 1]], ss[p]).wait()

    fire_lin(0, r0)
    fire_lin(1, r0 + 4)

    def body(gg, carry):
        g0 = 2 * gg
        wait_lin(0)
        fire_gat(0, sg[0])
        wait_lin(1)
        fire_gat(1, sg[1])
        wait_gat(0, sg[0])
        scale(0)
        fire_sct(0)
        wait_gat(1, sg[1])
        scale(1)
        fire_sct(1)
        wait_sct(0)
        fire_lin(0, jnp.minimum(r0 + (g0 + 2) * 4, rmax))
        wait_sct(1)
        fire_lin(1, jnp.minimum(r0 + (g0 + 3) * 4, rmax))
        return carry

    lax.fori_loop(0, nch // 2, body, 0)
    wait_lin(0)
    wait_lin(1)
    plsc.subcore_barrier()
    st = rows.at[0]
    for k in range(12):
        pltpu.sync_copy(acc.at[pl.ds(zoff + k * 512, 512)], st)
        pltpu.sync_copy(st, out_h.at[c, pl.ds(zoff + k * 512, 512)])
    pltpu.sync_copy(acc.at[pl.ds(zoff + 6144, 112)], rows.at[0, pl.ds(0, 112)])
    pltpu.sync_copy(rows.at[0, pl.ds(0, 112)],
                    out_h.at[c, pl.ds(zoff + 6144, 112)])


_WIDE_SCRATCH = [
    pltpu.VMEM((2, 4, 2, 128), i32),
    pltpu.VMEM((2, 4, 128), f32),
    pltpu.VMEM((2, 512, 16), f32),
    pltpu.VMEM_SHARED((N, 16), f32),
    pltpu.SemaphoreType.DMA,
    pltpu.SemaphoreType.DMA,
    pltpu.SemaphoreType.DMA,
    pltpu.SemaphoreType.DMA,
    pltpu.SemaphoreType.DMA,
    pltpu.SemaphoreType.DMA,
]


@functools.partial(
    pl.kernel,
    compiler_params=_params,
    out_type=jax.ShapeDtypeStruct((2, N, 16), f32),
    mesh=_mesh,
    scratch_types=list(_WIDE_SCRATCH),
)
def _prop16_kernel(ei_h, nrm_h, x_h, z2_h, out_h, eibuf, nbuf, rows, acc,
                   s0, s1, s2, s3, s4, s5):
    c = lax.axis_index("c")
    s = lax.axis_index("s")
    wid = c * 16 + s
    r0 = wid * 400

    def fire_gat(p, sem):
        for j in range(4):
            pltpu.async_copy(x_h.at[eibuf.at[p, j, 0]],
                             rows.at[p, pl.ds(j * 128, 128)], sem)

    def wait_gat(p, sem):
        for j in range(4):
            pltpu.make_async_copy(x_h.at[eibuf.at[p, j, 0]],
                                  rows.at[p, pl.ds(j * 128, 128)], sem).wait()

    _prop_wide_body(c, s, ei_h, nrm_h, out_h, z2_h, eibuf, nbuf, rows, acc,
                    (s0, s1, s2, s3, s4, s5), r0, r0 + 400 - 4, 100,
                    fire_gat, wait_gat)


@functools.partial(
    pl.kernel,
    compiler_params=_params,
    out_type=jax.ShapeDtypeStruct((2, N, 16), f32),
    mesh=_mesh,
    scratch_types=list(_WIDE_SCRATCH),
)
def _prop32_kernel(ei_h, nrm_h, x2_h, z2_h, out_h, eibuf, nbuf, rows, acc,
                   s0, s1, s2, s3, s4, s5):
    c = lax.axis_index("c")
    s = lax.axis_index("s")
    r0 = s * 800

    def fire_gat(p, sem):
        @pl.when(c == 0)
        def _g0():
            for j in range(4):
                pltpu.async_copy(x2_h.at[0].at[eibuf.at[p, j, 0]],
                                 rows.at[p, pl.ds(j * 128, 128)], sem)

        @pl.when(c == 1)
        def _g1():
            for j in range(4):
                pltpu.async_copy(x2_h.at[1].at[eibuf.at[p, j, 0]],
                                 rows.at[p, pl.ds(j * 128, 128)], sem)

    def wait_gat(p, sem):
        for j in range(4):
            pltpu.make_async_copy(x2_h.at[0].at[eibuf.at[p, j, 0]],
                                  rows.at[p, pl.ds(j * 128, 128)], sem).wait()

    _prop_wide_body(c, s, ei_h, nrm_h, out_h, z2_h, eibuf, nbuf, rows, acc,
                    (s0, s1, s2, s3, s4, s5), r0, r0 + 800 - 4, 200,
                    fire_gat, wait_gat)


def _dense(items, relu, split16=False, post_w=None):
    o = items[0][1].shape[-1]
    R = 2048
    nb = pl.cdiv(N, R)
    k = len(items)
    nw = k + (1 if post_w is not None else 0)

    def body(*refs):
        acc = None
        for j, (arr, w) in enumerate(items):
            h_ref, w_ref = refs[j], refs[k + j]
            if arr.ndim == 3:
                t = jnp.dot(h_ref[0], w_ref[0], preferred_element_type=f32,
                            precision=lax.Precision.HIGHEST)
                t = t + jnp.dot(h_ref[1], w_ref[1], preferred_element_type=f32,
                            precision=lax.Precision.HIGHEST)
            else:
                t = jnp.dot(h_ref[...], w_ref[...], preferred_element_type=f32,
                            precision=lax.Precision.HIGHEST)
            acc = t if acc is None else acc + t
        if relu:
            acc = jnp.maximum(acc, 0.0)
        outs = refs[k + nw:]
        if split16:
            outs[0][0, :, :] = acc[:, 0:16]
            outs[0][1, :, :] = acc[:, 16:32]
        else:
            outs[0][...] = acc
        if post_w is not None:
            outs[1][...] = jnp.dot(acc, refs[k + nw - 1][...],
                                   preferred_element_type=f32,
                            precision=lax.Precision.HIGHEST)

    in_specs = []
    for arr, _ in items:
        if arr.ndim == 3:
            in_specs.append(
                pl.BlockSpec((2, R, arr.shape[-1]), lambda i: (0, i, 0)))
        else:
            in_specs.append(pl.BlockSpec((R, arr.shape[-1]), lambda i: (i, 0)))
    w_ops = [w for _, w in items]
    if post_w is not None:
        w_ops = w_ops + [post_w]
    for w in w_ops:
        in_specs.append(
            pl.BlockSpec(w.shape, lambda i, nd=w.ndim: (0,) * nd))

    out_shapes = []
    out_specs = []
    if split16:
        out_shapes.append(jax.ShapeDtypeStruct((2, N, 16), f32))
        out_specs.append(pl.BlockSpec((2, R, 16), lambda i: (0, i, 0)))
    else:
        out_shapes.append(jax.ShapeDtypeStruct((N, o), f32))
        out_specs.append(pl.BlockSpec((R, o), lambda i: (i, 0)))
    if post_w is not None:
        out_shapes.append(jax.ShapeDtypeStruct((N, post_w.shape[-1]), f32))
        out_specs.append(
            pl.BlockSpec((R, post_w.shape[-1]), lambda i: (i, 0)))

    res = pl.pallas_call(
        body,
        grid=(nb,),
        in_specs=in_specs,
        out_specs=out_specs,
        out_shape=out_shapes,
    )(*[a for a, _ in items], *w_ops)
    return res if len(out_shapes) > 1 else res[0]


def _add2(p):
    P = N // 8
    pk = jnp.reshape(p, (2, P, 128))

    def body(p_ref, o_ref):
        o_ref[...] = p_ref[0] + p_ref[1]

    out = pl.pallas_call(
        body,
        out_shape=jax.ShapeDtypeStruct((P, 128), f32),
    )(pk)
    return jnp.reshape(out, (N, 16))


def kernel(x, edge_index, edge_attr, W1, W2, W3, Wc):
    n = x.shape[0]
    assert n == N and edge_index.shape[1] == E
    row, col = edge_index[0], edge_index[1]
    pad = EP - E
    pidx = (jnp.arange(pad, dtype=i32) * 997) % jnp.int32(n)
    srcp = jnp.concatenate([row, pidx]).reshape(M, 128)
    dstp = jnp.concatenate([col, pidx]).reshape(M, 128)
    ei2 = jnp.stack([srcp, dstp], axis=1)
    ewp = jnp.concatenate([edge_attr, jnp.zeros((pad,), f32)]).reshape(M, 128)

    normp = _norm_kernel(ei2, ewp)

    xv = x[:, 0]
    z1 = jnp.zeros((6256,), f32)
    z2 = jnp.zeros((3128, 16), f32)
    u1 = _prop1_kernel(ei2, normp, xv, z1)
    s1 = _prop1_kernel(ei2, normp, u1, z1)
    h1 = _dense(
        [(x, W1[0] - W1[2]), (u1[:, None], W1[1]), (s1[:, None], 2.0 * W1[2])],
        relu=True)

    u2p = _prop16_kernel(ei2, normp, h1, z2)
    u2 = _add2(u2p)
    s2p = _prop16_kernel(ei2, normp, u2, z2)
    C2 = 2.0 * W2[2]
    h2 = _dense(
        [(h1, W2[0] - W2[2]), (u2, W2[1]), (s2p, jnp.stack([C2, C2]))],
        relu=True, split16=True)

    u3 = _prop32_kernel(ei2, normp, h2, z2)
    s3 = _prop32_kernel(ei2, normp, u3, z2)
    A3 = W3[0] - W3[2]
    B3 = W3[1]
    C3 = 2.0 * W3[2]
    P = jnp.concatenate([Wc[1], Wc[2], jnp.zeros((64, 12), f32)], axis=1)
    h3, ab = _dense(
        [(h2, A3.reshape(2, 16, 64)), (u3, B3.reshape(2, 16, 64)),
         (s3, C3.reshape(2, 16, 64))],
        relu=True, post_w=P)

    qp = _prop16_kernel(ei2, normp, ab, z2)
    q = _add2(qp)
    rp = _prop16_kernel(ei2, normp, q, z2)
    S1 = jnp.zeros((16, 2), f32).at[0, 0].set(1.0).at[1, 1].set(1.0)
    S2 = jnp.zeros((16, 2), f32).at[2, 0].set(2.0).at[3, 1].set(2.0)
    out = _dense([(h3, Wc[0] - Wc[2]), (q, S1), (rp, jnp.stack([S2, S2]))],
                 relu=False)
    return out


# speedup vs baseline: 1.4936x; 1.4936x over previous
import functools

import jax
import jax.numpy as jnp
from jax import lax
from jax.experimental import pallas as pl
from jax.experimental.pallas import tpu as pltpu
from jax.experimental.pallas import tpu_sc as plsc

f32 = jnp.float32
i32 = jnp.int32

N = 100000
NP = 102400
P8 = NP // 8
E = 1600000
EP = 1638400
M = EP // 128

_mesh = plsc.VectorSubcoreMesh(core_axis_name="c", subcore_axis_name="s")
_params = pltpu.CompilerParams(needs_layout_passes=False,
                               use_tc_tiling_on_sc=False)


def _nr_rsqrt(x):
    i = lax.bitcast_convert_type(x, i32)
    i = jnp.int32(0x5F3759DF) - lax.shift_right_arithmetic(
        i, jnp.full((16,), 1, i32))
    y = lax.bitcast_convert_type(i, f32)
    for _ in range(3):
        y = y * (1.5 - 0.5 * x * y * y)
    return jnp.where(x > 0.0, y, jnp.zeros_like(y))


@functools.partial(
    pl.kernel,
    compiler_params=_params,
    out_type=jax.ShapeDtypeStruct((M, 128), f32),
    mesh=_mesh,
    scratch_types=[
        pltpu.VMEM((2, 8, 2, 128), i32),
        pltpu.VMEM((2, 8, 128), f32),
        pltpu.VMEM((2, 8, 128), f32),
        pltpu.VMEM((NP,), f32),
        pltpu.VMEM((6400,), f32),
        pltpu.VMEM_SHARED((NP,), f32),
        pltpu.SemaphoreType.DMA,
        pltpu.SemaphoreType.DMA,
        pltpu.SemaphoreType.DMA,
        pltpu.SemaphoreType.DMA,
    ],
)
def _norm_kernel(ei_h, ew_h, out_h, eibuf, ebuf, mbuf, dx, tbuf, dacc,
                 sl0, sl1, sx0, sx1):
    c = lax.axis_index("c")
    s = lax.axis_index("s")
    zoff = s * 6400
    sl = (sl0, sl1)
    sx = (sx0, sx1)

    def fire_lin(p, b):
        pltpu.async_copy(ei_h.at[pl.ds(b, 8)], eibuf.at[p], sl[p])
        pltpu.async_copy(ew_h.at[pl.ds(b, 8)], ebuf.at[p], sl[p])

    def wait_lin(p):
        pltpu.make_async_copy(ei_h.at[pl.ds(0, 8)], eibuf.at[p], sl[p]).wait()
        pltpu.make_async_copy(ew_h.at[pl.ds(0, 8)], ebuf.at[p], sl[p]).wait()

    def _zb(i, carry):
        tbuf[pl.ds(i * 16, 16)] = jnp.zeros((16,), f32)
        return carry

    lax.fori_loop(0, 6400 // 16, _zb, 0)
    pltpu.sync_copy(tbuf, dacc.at[pl.ds(zoff, 6400)])
    plsc.subcore_barrier()

    ra0 = s * 800
    ramax = ra0 + 800 - 8

    def comp_a(p):
        for j in range(8):
            @plsc.parallel_loop(0, 128, 16, unroll=2)
            def _mk(e):
                sv = eibuf[p, j, 0, pl.ds(e, 16)]
                dv = eibuf[p, j, 1, pl.ds(e, 16)]
                ev = ebuf[p, j, pl.ds(e, 16)]
                mbuf[p, j, pl.ds(e, 16)] = jnp.where(sv == dv, 0.0, ev)

    def fire_sct(p):
        for j in range(8):
            pltpu.async_copy(mbuf.at[p, j], dacc.at[eibuf.at[p, j, 0]],
                             sx[p], add=True)

    def wait_sct(p):
        for j in range(8):
            pltpu.make_async_copy(mbuf.at[p, j], dacc.at[eibuf.at[p, j, 0]],
                                  sx[p]).wait()

    fire_lin(0, ra0)
    fire_lin(1, ra0 + 8)

    def body_a(gg, carry):
        g0 = 2 * gg
        wait_lin(0)
        comp_a(0)
        fire_sct(0)
        wait_lin(1)
        comp_a(1)
        fire_sct(1)
        wait_sct(0)
        fire_lin(0, jnp.minimum(ra0 + (g0 + 2) * 8, ramax))
        wait_sct(1)
        fire_lin(1, jnp.minimum(ra0 + (g0 + 3) * 8, ramax))
        return carry

    lax.fori_loop(0, 50, body_a, 0)
    wait_lin(0)
    wait_lin(1)
    plsc.subcore_barrier()

    pltpu.sync_copy(dacc.at[pl.ds(zoff, 6400)], tbuf)
    plsc.subcore_barrier()

    @plsc.parallel_loop(0, 6400, 16, unroll=2)
    def _rs(i):
        tbuf[pl.ds(i, 16)] = _nr_rsqrt(tbuf[pl.ds(i, 16)])

    pltpu.sync_copy(tbuf, dacc.at[pl.ds(zoff, 6400)])
    plsc.subcore_barrier()
    pltpu.sync_copy(dacc, dx)

    rc0 = c * 6400 + s * 400

    def comp_c(p):
        for j in range(8):
            @plsc.parallel_loop(0, 128, 16, unroll=2)
            def _mk(e):
                sv = eibuf[p, j, 0, pl.ds(e, 16)]
                dv = eibuf[p, j, 1, pl.ds(e, 16)]
                ev = ebuf[p, j, pl.ds(e, 16)]
                a = plsc.load_gather(dx, [sv])
                b = plsc.load_gather(dx, [dv])
                ewp = jnp.where(sv == dv, 0.0, ev)
                mbuf[p, j, pl.ds(e, 16)] = (-a) * ewp * b

    def fire_out(p, b):
        pltpu.async_copy(mbuf.at[p], out_h.at[pl.ds(b, 8)], sx[p])

    def wait_out(p):
        pltpu.make_async_copy(mbuf.at[p], out_h.at[pl.ds(0, 8)], sx[p]).wait()

    rcmax = rc0 + 400 - 8
    fire_lin(0, rc0)
    fire_lin(1, rc0 + 8)

    def body_c(gg, carry):
        g0 = 2 * gg
        wait_lin(0)
        comp_c(0)
        fire_out(0, rc0 + g0 * 8)
        wait_lin(1)
        comp_c(1)
        fire_out(1, rc0 + (g0 + 1) * 8)
        wait_out(0)
        fire_lin(0, jnp.minimum(rc0 + (g0 + 2) * 8, rcmax))
        wait_out(1)
        fire_lin(1, jnp.minimum(rc0 + (g0 + 3) * 8, rcmax))
        return carry

    lax.fori_loop(0, 25, body_c, 0)
    wait_lin(0)
    wait_lin(1)


@functools.partial(
    pl.kernel,
    compiler_params=_params,
    out_type=jax.ShapeDtypeStruct((NP,), f32),
    mesh=_mesh,
    scratch_types=[
        pltpu.VMEM((2, 8, 2, 128), i32),
        pltpu.VMEM((2, 8, 128), f32),
        pltpu.VMEM((2, 8, 128), f32),
        pltpu.VMEM((NP,), f32),
        pltpu.VMEM_SHARED((NP,), f32),
        pltpu.SemaphoreType.DMA,
        pltpu.SemaphoreType.DMA,
        pltpu.SemaphoreType.DMA,
        pltpu.SemaphoreType.DMA,
    ],
)
def _prop1_kernel(ei_h, nrm_h, x_h, z1_h, out_h, eibuf, nbuf, mbuf, xbuf, acc,
                  sl0, sl1, sx0, sx1):
    c = lax.axis_index("c")
    s = lax.axis_index("s")
    zoff = s * 6400
    sl = (sl0, sl1)
    sx = (sx0, sx1)
    pltpu.sync_copy(z1_h, acc.at[pl.ds(zoff, 6400)])
    pltpu.sync_copy(x_h, xbuf)
    plsc.subcore_barrier()

    r0 = s * 800
    rmax = r0 + 800 - 8

    def fire_lin(p, b):
        pltpu.async_copy(ei_h.at[pl.ds(b, 8)], eibuf.at[p], sl[p])
        pltpu.async_copy(nrm_h.at[pl.ds(b, 8)], nbuf.at[p], sl[p])

    def wait_lin(p):
        pltpu.make_async_copy(ei_h.at[pl.ds(0, 8)], eibuf.at[p], sl[p]).wait()
        pltpu.make_async_copy(nrm_h.at[pl.ds(0, 8)], nbuf.at[p], sl[p]).wait()

    def comp(p):
        for j in range(8):
            @plsc.parallel_loop(0, 128, 16, unroll=2)
            def _mk(e):
                sv = eibuf[p, j, 0, pl.ds(e, 16)]
                xg = plsc.load_gather(xbuf, [sv])
                mbuf[p, j, pl.ds(e, 16)] = xg * nbuf[p, j, pl.ds(e, 16)]

    def fire_sct(p):
        for j in range(8):
            pltpu.async_copy(mbuf.at[p, j], acc.at[eibuf.at[p, j, 1]],
                             sx[p], add=True)

    def wait_sct(p):
        for j in range(8):
            pltpu.make_async_copy(mbuf.at[p, j], acc.at[eibuf.at[p, j, 1]],
                                  sx[p]).wait()

    fire_lin(0, r0)
    fire_lin(1, r0 + 8)

    def body(gg, carry):
        g0 = 2 * gg
        wait_lin(0)
        comp(0)
        fire_sct(0)
        wait_lin(1)
        comp(1)
        fire_sct(1)
        wait_sct(0)
        fire_lin(0, jnp.minimum(r0 + (g0 + 2) * 8, rmax))
        wait_sct(1)
        fire_lin(1, jnp.minimum(r0 + (g0 + 3) * 8, rmax))
        return carry

    lax.fori_loop(0, 50, body, 0)
    wait_lin(0)
    wait_lin(1)
    plsc.subcore_barrier()
    off = c * 51200 + s * 3200
    pltpu.sync_copy(acc.at[pl.ds(off, 3200)], xbuf.at[pl.ds(0, 3200)])
    pltpu.sync_copy(xbuf.at[pl.ds(0, 3200)], out_h.at[pl.ds(off, 3200)])


def _prop_wide_body(c, s, ei_h, nrm_h, out_h, z2_h, eibuf, nbuf, rows, acc,
                    sems, r0, rmax, nch, fire_gat, wait_gat):
    sl = (sems[0], sems[1])
    sg = (sems[2], sems[3])
    ss = (sems[4], sems[5])
    zoff = s * 6400
    for k in range(2):
        pltpu.sync_copy(z2_h, acc.at[pl.ds(zoff + k * 3200, 3200)])
    plsc.subcore_barrier()

    def fire_lin(p, b):
        pltpu.async_copy(ei_h.at[pl.ds(b, 4)], eibuf.at[p], sl[p])
        pltpu.async_copy(nrm_h.at[pl.ds(b, 4)], nbuf.at[p], sl[p])

    def wait_lin(p):
        pltpu.make_async_copy(ei_h.at[pl.ds(0, 4)], eibuf.at[p], sl[p]).wait()
        pltpu.make_async_copy(nrm_h.at[pl.ds(0, 4)], nbuf.at[p], sl[p]).wait()

    def scale(p):
        for j in range(4):
            @plsc.parallel_loop(0, 128, 16)
            def _sc(e):
                nv = nbuf[p, j, pl.ds(e, 16)]
                for kk in range(16):
                    idx = j * 128 + e + kk
                    rows[p, idx, :] = rows[p, idx, :] * nv[kk]

    def fire_sct(p):
        for j in range(4):
            pltpu.async_copy(rows.at[p, pl.ds(j * 128, 128)],
                             acc.at[eibuf.at[p, j, 1]], ss[p], add=True)

    def wait_sct(p):
        for j in range(4):
            pltpu.make_async_copy(rows.at[p, pl.ds(j * 128, 128)],
                                  acc.at[eibuf.at[p, j, 1]], ss[p]).wait()

    fire_lin(0, r0)
    fire_lin(1, r0 + 4)

    def body(gg, carry):
        g0 = 2 * gg
        wait_lin(0)
        fire_gat(0, sg[0])
        wait_lin(1)
        fire_gat(1, sg[1])
        wait_gat(0, sg[0])
        scale(0)
        fire_sct(0)
        wait_gat(1, sg[1])
        scale(1)
        fire_sct(1)
        wait_sct(0)
        fire_lin(0, jnp.minimum(r0 + (g0 + 2) * 4, rmax))
        wait_sct(1)
        fire_lin(1, jnp.minimum(r0 + (g0 + 3) * 4, rmax))
        return carry

    lax.fori_loop(0, nch // 2, body, 0)
    wait_lin(0)
    wait_lin(1)
    plsc.subcore_barrier()
    st = rows.at[0]
    for k in range(12):
        pltpu.sync_copy(acc.at[pl.ds(zoff + k * 512, 512)], st)
        pltpu.sync_copy(st, out_h.at[c, pl.ds(zoff + k * 512, 512)])
    pltpu.sync_copy(acc.at[pl.ds(zoff + 6144, 256)], rows.at[0, pl.ds(0, 256)])
    pltpu.sync_copy(rows.at[0, pl.ds(0, 256)],
                    out_h.at[c, pl.ds(zoff + 6144, 256)])


_WIDE_SCRATCH = [
    pltpu.VMEM((2, 4, 2, 128), i32),
    pltpu.VMEM((2, 4, 128), f32),
    pltpu.VMEM((2, 512, 16), f32),
    pltpu.VMEM_SHARED((NP, 16), f32),
    pltpu.SemaphoreType.DMA,
    pltpu.SemaphoreType.DMA,
    pltpu.SemaphoreType.DMA,
    pltpu.SemaphoreType.DMA,
    pltpu.SemaphoreType.DMA,
    pltpu.SemaphoreType.DMA,
]


@functools.partial(
    pl.kernel,
    compiler_params=_params,
    out_type=jax.ShapeDtypeStruct((2, NP, 16), f32),
    mesh=_mesh,
    scratch_types=list(_WIDE_SCRATCH),
)
def _prop16_kernel(ei_h, nrm_h, x_h, z2_h, out_h, eibuf, nbuf, rows, acc,
                   s0, s1, s2, s3, s4, s5):
    c = lax.axis_index("c")
    s = lax.axis_index("s")
    wid = c * 16 + s
    r0 = wid * 400

    def fire_gat(p, sem):
        for j in range(4):
            pltpu.async_copy(x_h.at[eibuf.at[p, j, 0]],
                             rows.at[p, pl.ds(j * 128, 128)], sem)

    def wait_gat(p, sem):
        for j in range(4):
            pltpu.make_async_copy(x_h.at[eibuf.at[p, j, 0]],
                                  rows.at[p, pl.ds(j * 128, 128)], sem).wait()

    _prop_wide_body(c, s, ei_h, nrm_h, out_h, z2_h, eibuf, nbuf, rows, acc,
                    (s0, s1, s2, s3, s4, s5), r0, r0 + 400 - 4, 100,
                    fire_gat, wait_gat)


@functools.partial(
    pl.kernel,
    compiler_params=_params,
    out_type=jax.ShapeDtypeStruct((2, NP, 16), f32),
    mesh=_mesh,
    scratch_types=list(_WIDE_SCRATCH),
)
def _prop32_kernel(ei_h, nrm_h, x2_h, z2_h, out_h, eibuf, nbuf, rows, acc,
                   s0, s1, s2, s3, s4, s5):
    c = lax.axis_index("c")
    s = lax.axis_index("s")
    r0 = s * 800

    def fire_gat(p, sem):
        @pl.when(c == 0)
        def _g0():
            for j in range(4):
                pltpu.async_copy(x2_h.at[0].at[eibuf.at[p, j, 0]],
                                 rows.at[p, pl.ds(j * 128, 128)], sem)

        @pl.when(c == 1)
        def _g1():
            for j in range(4):
                pltpu.async_copy(x2_h.at[1].at[eibuf.at[p, j, 0]],
                                 rows.at[p, pl.ds(j * 128, 128)], sem)

    def wait_gat(p, sem):
        for j in range(4):
            pltpu.make_async_copy(x2_h.at[0].at[eibuf.at[p, j, 0]],
                                  rows.at[p, pl.ds(j * 128, 128)], sem).wait()

    _prop_wide_body(c, s, ei_h, nrm_h, out_h, z2_h, eibuf, nbuf, rows, acc,
                    (s0, s1, s2, s3, s4, s5), r0, r0 + 800 - 4, 200,
                    fire_gat, wait_gat)


def _bd(w):
    return jnp.kron(jnp.eye(8, dtype=f32), w)


def _pdense(items, relu, post_w=None):
    o = items[0][1].shape[-1]
    G = o // 16 if o >= 16 else 1
    ow = 128 if o >= 16 else 8 * o
    R = 1600
    nb = P8 // R
    k = len(items)

    wexp = []
    for arr, w in items:
        for g in range(G):
            blk = w[..., 16 * g:16 * (g + 1)] if o >= 16 else w
            if arr.ndim == 3:
                wexp.append(jnp.stack([_bd(blk[t])
                                       for t in range(arr.shape[0])]))
            else:
                wexp.append(_bd(blk))
    if post_w is not None:
        wexp.append(jnp.stack([_bd(post_w[16 * g:16 * (g + 1), :])
                               for g in range(G)]))
    nw = len(wexp)

    def body(*refs):
        ins = refs[:k]
        ws = refs[k:k + nw]
        outs = refs[k + nw:]
        accs = []
        for g in range(G):
            acc = None
            for j, (arr, _) in enumerate(items):
                wr = ws[j * G + g]
                if arr.ndim == 3:
                    t = None
                    for tc in range(arr.shape[0]):
                        d = jnp.dot(ins[j][tc], wr[tc],
                                    preferred_element_type=f32,
                                    precision=lax.Precision.HIGHEST)
                        t = d if t is None else t + d
                else:
                    t = jnp.dot(ins[j][...], wr[...],
                                preferred_element_type=f32,
                                precision=lax.Precision.HIGHEST)
                acc = t if acc is None else acc + t
            if relu:
                acc = jnp.maximum(acc, 0.0)
            accs.append(acc)
        if G > 1:
            for g in range(G):
                outs[0][g, :, :] = accs[g]
        else:
            outs[0][...] = accs[0]
        if post_w is not None:
            pr = ws[k * G]
            pa = None
            for g in range(G):
                d = jnp.dot(accs[g], pr[g], preferred_element_type=f32,
                            precision=lax.Precision.HIGHEST)
                pa = d if pa is None else pa + d
            outs[1][...] = pa

    in_specs = []
    for arr, _ in items:
        if arr.ndim == 3:
            in_specs.append(pl.BlockSpec(
                (arr.shape[0], R, arr.shape[-1]),
                lambda i: (0, i, 0)))
        else:
            in_specs.append(
                pl.BlockSpec((R, arr.shape[-1]), lambda i: (i, 0)))
    for w in wexp:
        in_specs.append(
            pl.BlockSpec(w.shape, lambda i, nd=w.ndim: (0,) * nd))

    out_shapes = []
    out_specs = []
    if G > 1:
        out_shapes.append(jax.ShapeDtypeStruct((G, P8, 128), f32))
        out_specs.append(pl.BlockSpec((G, R, 128), lambda i: (0, i, 0)))
    else:
        out_shapes.append(jax.ShapeDtypeStruct((P8, ow), f32))
        out_specs.append(pl.BlockSpec((R, ow), lambda i: (i, 0)))
    if post_w is not None:
        out_shapes.append(jax.ShapeDtypeStruct((P8, 128), f32))
        out_specs.append(pl.BlockSpec((R, 128), lambda i: (i, 0)))

    res = pl.pallas_call(
        body,
        grid=(nb,),
        in_specs=in_specs,
        out_specs=out_specs,
        out_shape=out_shapes,
    )(*[a for a, _ in items], *wexp)
    return res if len(out_shapes) > 1 else res[0]


def _add2(pk):
    R = 1600
    nb = P8 // R

    def body(p_ref, o_ref):
        o_ref[...] = p_ref[0] + p_ref[1]

    return pl.pallas_call(
        body,
        grid=(nb,),
        in_specs=[pl.BlockSpec((2, R, 128), lambda i: (0, i, 0))],
        out_specs=pl.BlockSpec((R, 128), lambda i: (i, 0)),
        out_shape=jax.ShapeDtypeStruct((P8, 128), f32),
    )(pk)


def kernel(x, edge_index, edge_attr, W1, W2, W3, Wc):
    n = x.shape[0]
    assert n == N and edge_index.shape[1] == E
    row, col = edge_index[0], edge_index[1]
    pad = EP - E
    pidx = (jnp.arange(pad, dtype=i32) * 997) % jnp.int32(n)
    srcp = jnp.concatenate([row, pidx]).reshape(M, 128)
    dstp = jnp.concatenate([col, pidx]).reshape(M, 128)
    ei2 = jnp.stack([srcp, dstp], axis=1)
    ewp = jnp.concatenate([edge_attr, jnp.zeros((pad,), f32)]).reshape(M, 128)

    normp = _norm_kernel(ei2, ewp)

    xv = jnp.pad(x[:, 0], (0, NP - N))
    z1 = jnp.zeros((6400,), f32)
    z2 = jnp.zeros((3200, 16), f32)
    u1 = _prop1_kernel(ei2, normp, xv, z1)
    s1 = _prop1_kernel(ei2, normp, u1, z1)
    h1p = _pdense(
        [(jnp.reshape(xv, (P8, 8)), W1[0] - W1[2]),
         (jnp.reshape(u1, (P8, 8)), W1[1]),
         (jnp.reshape(s1, (P8, 8)), 2.0 * W1[2])],
        relu=True)
    h1 = jnp.reshape(h1p, (NP, 16))

    u2p = _prop16_kernel(ei2, normp, h1, z2)
    u2k = _add2(jnp.reshape(u2p, (2, P8, 128)))
    u2 = jnp.reshape(u2k, (NP, 16))
    s2p = _prop16_kernel(ei2, normp, u2, z2)
    C2 = 2.0 * W2[2]
    h2k = _pdense(
        [(h1p, W2[0] - W2[2]), (u2k, W2[1]),
         (jnp.reshape(s2p, (2, P8, 128)), jnp.stack([C2, C2]))],
        relu=True)
    h2 = jnp.reshape(h2k, (2, NP, 16))

    u3 = _prop32_kernel(ei2, normp, h2, z2)
    s3 = _prop32_kernel(ei2, normp, u3, z2)
    A3 = (W3[0] - W3[2]).reshape(2, 16, 64)
    B3 = W3[1].reshape(2, 16, 64)
    C3 = (2.0 * W3[2]).reshape(2, 16, 64)
    P = jnp.concatenate([Wc[1], Wc[2], jnp.zeros((64, 12), f32)], axis=1)
    h3k, abk = _pdense(
        [(h2k, A3), (jnp.reshape(u3, (2, P8, 128)), B3),
         (jnp.reshape(s3, (2, P8, 128)), C3)],
        relu=True, post_w=P)

    ab = jnp.reshape(abk, (NP, 16))
    qp = _prop16_kernel(ei2, normp, ab, z2)
    qk = _add2(jnp.reshape(qp, (2, P8, 128)))
    q = jnp.reshape(qk, (NP, 16))
    rp = _prop16_kernel(ei2, normp, q, z2)
    D4 = (Wc[0] - Wc[2]).reshape(4, 16, 2)
    S1 = jnp.zeros((16, 2), f32).at[0, 0].set(1.0).at[1, 1].set(1.0)
    S2 = jnp.zeros((16, 2), f32).at[2, 0].set(2.0).at[3, 1].set(2.0)
    outp = _pdense(
        [(h3k, D4), (qk, S1),
         (jnp.reshape(rp, (2, P8, 128)), jnp.stack([S2, S2]))],
        relu=False)
    return jnp.reshape(outp, (NP, 2))[:N]


# speedup vs baseline: 1.5283x; 1.0233x over previous
import functools

import jax
import jax.numpy as jnp
from jax import lax
from jax.experimental import pallas as pl
from jax.experimental.pallas import tpu as pltpu
from jax.experimental.pallas import tpu_sc as plsc

f32 = jnp.float32
i32 = jnp.int32

N = 100000
NP = 102400
P8 = NP // 8
E = 1600000
EP = 1638400
M = EP // 128

_mesh = plsc.VectorSubcoreMesh(core_axis_name="c", subcore_axis_name="s")
_params = pltpu.CompilerParams(needs_layout_passes=False,
                               use_tc_tiling_on_sc=False)


def _nr_rsqrt(x):
    i = lax.bitcast_convert_type(x, i32)
    i = jnp.int32(0x5F3759DF) - lax.shift_right_arithmetic(
        i, jnp.full((16,), 1, i32))
    y = lax.bitcast_convert_type(i, f32)
    for _ in range(3):
        y = y * (1.5 - 0.5 * x * y * y)
    return jnp.where(x > 0.0, y, jnp.zeros_like(y))


@functools.partial(
    pl.kernel,
    compiler_params=_params,
    out_type=jax.ShapeDtypeStruct((M, 128), f32),
    mesh=_mesh,
    scratch_types=[
        pltpu.VMEM((2, 8, 2, 128), i32),
        pltpu.VMEM((2, 8, 128), f32),
        pltpu.VMEM((2, 8, 128), f32),
        pltpu.VMEM((NP,), f32),
        pltpu.VMEM((6400,), f32),
        pltpu.VMEM_SHARED((NP,), f32),
        pltpu.SemaphoreType.DMA,
        pltpu.SemaphoreType.DMA,
        pltpu.SemaphoreType.DMA,
        pltpu.SemaphoreType.DMA,
    ],
)
def _norm_kernel(ei_h, ew_h, out_h, eibuf, ebuf, mbuf, dx, tbuf, dacc,
                 sl0, sl1, sx0, sx1):
    c = lax.axis_index("c")
    s = lax.axis_index("s")
    zoff = s * 6400
    sl = (sl0, sl1)
    sx = (sx0, sx1)

    def fire_lin(p, b):
        pltpu.async_copy(ei_h.at[pl.ds(b, 8)], eibuf.at[p], sl[p])
        pltpu.async_copy(ew_h.at[pl.ds(b, 8)], ebuf.at[p], sl[p])

    def wait_lin(p):
        pltpu.make_async_copy(ei_h.at[pl.ds(0, 8)], eibuf.at[p], sl[p]).wait()
        pltpu.make_async_copy(ew_h.at[pl.ds(0, 8)], ebuf.at[p], sl[p]).wait()

    def _zb(i, carry):
        tbuf[pl.ds(i * 16, 16)] = jnp.zeros((16,), f32)
        return carry

    lax.fori_loop(0, 6400 // 16, _zb, 0)
    pltpu.sync_copy(tbuf, dacc.at[pl.ds(zoff, 6400)])
    plsc.subcore_barrier()

    ra0 = s * 800
    ramax = ra0 + 800 - 8

    def comp_a(p):
        for j in range(8):
            @plsc.parallel_loop(0, 128, 16, unroll=2)
            def _mk(e):
                sv = eibuf[p, j, 0, pl.ds(e, 16)]
                dv = eibuf[p, j, 1, pl.ds(e, 16)]
                ev = ebuf[p, j, pl.ds(e, 16)]
                mbuf[p, j, pl.ds(e, 16)] = jnp.where(sv == dv, 0.0, ev)

    def fire_sct(p):
        for j in range(8):
            pltpu.async_copy(mbuf.at[p, j], dacc.at[eibuf.at[p, j, 0]],
                             sx[p], add=True)

    def wait_sct(p):
        for j in range(8):
            pltpu.make_async_copy(mbuf.at[p, j], dacc.at[eibuf.at[p, j, 0]],
                                  sx[p]).wait()

    fire_lin(0, ra0)
    fire_lin(1, ra0 + 8)

    def body_a(gg, carry):
        g0 = 2 * gg
        wait_lin(0)
        comp_a(0)
        fire_sct(0)
        wait_lin(1)
        comp_a(1)
        fire_sct(1)
        wait_sct(0)
        fire_lin(0, jnp.minimum(ra0 + (g0 + 2) * 8, ramax))
        wait_sct(1)
        fire_lin(1, jnp.minimum(ra0 + (g0 + 3) * 8, ramax))
        return carry

    lax.fori_loop(0, 50, body_a, 0)
    wait_lin(0)
    wait_lin(1)
    plsc.subcore_barrier()

    pltpu.sync_copy(dacc.at[pl.ds(zoff, 6400)], tbuf)
    plsc.subcore_barrier()

    @plsc.parallel_loop(0, 6400, 16, unroll=2)
    def _rs(i):
        tbuf[pl.ds(i, 16)] = _nr_rsqrt(tbuf[pl.ds(i, 16)])

    pltpu.sync_copy(tbuf, dacc.at[pl.ds(zoff, 6400)])
    plsc.subcore_barrier()
    pltpu.sync_copy(dacc, dx)

    rc0 = c * 6400 + s * 400

    def comp_c(p):
        for j in range(8):
            @plsc.parallel_loop(0, 128, 16, unroll=2)
            def _mk(e):
                sv = eibuf[p, j, 0, pl.ds(e, 16)]
                dv = eibuf[p, j, 1, pl.ds(e, 16)]
                ev = ebuf[p, j, pl.ds(e, 16)]
                a = plsc.load_gather(dx, [sv])
                b = plsc.load_gather(dx, [dv])
                ewp = jnp.where(sv == dv, 0.0, ev)
                mbuf[p, j, pl.ds(e, 16)] = (-a) * ewp * b

    def fire_out(p, b):
        pltpu.async_copy(mbuf.at[p], out_h.at[pl.ds(b, 8)], sx[p])

    def wait_out(p):
        pltpu.make_async_copy(mbuf.at[p], out_h.at[pl.ds(0, 8)], sx[p]).wait()

    rcmax = rc0 + 400 - 8
    fire_lin(0, rc0)
    fire_lin(1, rc0 + 8)

    def body_c(gg, carry):
        g0 = 2 * gg
        wait_lin(0)
        comp_c(0)
        fire_out(0, rc0 + g0 * 8)
        wait_lin(1)
        comp_c(1)
        fire_out(1, rc0 + (g0 + 1) * 8)
        wait_out(0)
        fire_lin(0, jnp.minimum(rc0 + (g0 + 2) * 8, rcmax))
        wait_out(1)
        fire_lin(1, jnp.minimum(rc0 + (g0 + 3) * 8, rcmax))
        return carry

    lax.fori_loop(0, 25, body_c, 0)
    wait_lin(0)
    wait_lin(1)


@functools.partial(
    pl.kernel,
    compiler_params=_params,
    out_type=jax.ShapeDtypeStruct((NP,), f32),
    mesh=_mesh,
    scratch_types=[
        pltpu.VMEM((2, 8, 2, 128), i32),
        pltpu.VMEM((2, 8, 128), f32),
        pltpu.VMEM((2, 8, 128), f32),
        pltpu.VMEM((NP,), f32),
        pltpu.VMEM_SHARED((NP,), f32),
        pltpu.SemaphoreType.DMA,
        pltpu.SemaphoreType.DMA,
        pltpu.SemaphoreType.DMA,
        pltpu.SemaphoreType.DMA,
    ],
)
def _prop1_kernel(ei_h, nrm_h, x_h, z1_h, out_h, eibuf, nbuf, mbuf, xbuf, acc,
                  sl0, sl1, sx0, sx1):
    c = lax.axis_index("c")
    s = lax.axis_index("s")
    zoff = s * 6400
    sl = (sl0, sl1)
    sx = (sx0, sx1)
    pltpu.sync_copy(z1_h, acc.at[pl.ds(zoff, 6400)])
    pltpu.sync_copy(x_h, xbuf)
    plsc.subcore_barrier()

    r0 = s * 800
    rmax = r0 + 800 - 8

    def fire_lin(p, b):
        pltpu.async_copy(ei_h.at[pl.ds(b, 8)], eibuf.at[p], sl[p])
        pltpu.async_copy(nrm_h.at[pl.ds(b, 8)], nbuf.at[p], sl[p])

    def wait_lin(p):
        pltpu.make_async_copy(ei_h.at[pl.ds(0, 8)], eibuf.at[p], sl[p]).wait()
        pltpu.make_async_copy(nrm_h.at[pl.ds(0, 8)], nbuf.at[p], sl[p]).wait()

    def comp(p):
        for j in range(8):
            @plsc.parallel_loop(0, 128, 16, unroll=2)
            def _mk(e):
                sv = eibuf[p, j, 0, pl.ds(e, 16)]
                xg = plsc.load_gather(xbuf, [sv])
                mbuf[p, j, pl.ds(e, 16)] = xg * nbuf[p, j, pl.ds(e, 16)]

    def fire_sct(p):
        for j in range(8):
            pltpu.async_copy(mbuf.at[p, j], acc.at[eibuf.at[p, j, 1]],
                             sx[p], add=True)

    def wait_sct(p):
        for j in range(8):
            pltpu.make_async_copy(mbuf.at[p, j], acc.at[eibuf.at[p, j, 1]],
                                  sx[p]).wait()

    fire_lin(0, r0)
    fire_lin(1, r0 + 8)

    def body(gg, carry):
        g0 = 2 * gg
        wait_lin(0)
        comp(0)
        fire_sct(0)
        wait_lin(1)
        comp(1)
        fire_sct(1)
        wait_sct(0)
        fire_lin(0, jnp.minimum(r0 + (g0 + 2) * 8, rmax))
        wait_sct(1)
        fire_lin(1, jnp.minimum(r0 + (g0 + 3) * 8, rmax))
        return carry

    lax.fori_loop(0, 50, body, 0)
    wait_lin(0)
    wait_lin(1)
    plsc.subcore_barrier()
    off = c * 51200 + s * 3200
    pltpu.sync_copy(acc.at[pl.ds(off, 3200)], xbuf.at[pl.ds(0, 3200)])
    pltpu.sync_copy(xbuf.at[pl.ds(0, 3200)], out_h.at[pl.ds(off, 3200)])


def _prop_wide_body(c, s, ei_h, nrm_h, out_h, z2_h, eibuf, nbuf, rows, acc,
                    sems, r0, rmax, nch, fire_gat, wait_gat):
    sl = (sems[0], sems[1])
    sg = (sems[2], sems[3])
    ss = (sems[4], sems[5])
    zoff = s * 6400
    for k in range(2):
        pltpu.sync_copy(z2_h, acc.at[pl.ds(zoff + k * 3200, 3200)])
    plsc.subcore_barrier()

    def fire_lin(p, b):
        pltpu.async_copy(ei_h.at[pl.ds(b, 4)], eibuf.at[p], sl[p])
        pltpu.async_copy(nrm_h.at[pl.ds(b, 4)], nbuf.at[p], sl[p])

    def wait_lin(p):
        pltpu.make_async_copy(ei_h.at[pl.ds(0, 4)], eibuf.at[p], sl[p]).wait()
        pltpu.make_async_copy(nrm_h.at[pl.ds(0, 4)], nbuf.at[p], sl[p]).wait()

    def scale(p):
        for j in range(4):
            @plsc.parallel_loop(0, 128, 16, unroll=2)
            def _sc(e):
                nv = nbuf[p, j, pl.ds(e, 16)]
                for kk in range(16):
                    idx = j * 128 + e + kk
                    rows[p, idx, :] = rows[p, idx, :] * nv[kk]

    def fire_sct(p):
        for j in range(4):
            pltpu.async_copy(rows.at[p, pl.ds(j * 128, 128)],
                             acc.at[eibuf.at[p, j, 1]], ss[p], add=True)

    def wait_sct(p):
        for j in range(4):
            pltpu.make_async_copy(rows.at[p, pl.ds(j * 128, 128)],
                                  acc.at[eibuf.at[p, j, 1]], ss[p]).wait()

    fire_lin(0, r0)
    fire_lin(1, r0 + 4)

    def body(gg, carry):
        g0 = 2 * gg
        wait_lin(0)
        fire_gat(0, sg[0])
        wait_lin(1)
        fire_gat(1, sg[1])
        wait_gat(0, sg[0])
        scale(0)
        fire_sct(0)
        wait_gat(1, sg[1])
        scale(1)
        fire_sct(1)
        wait_sct(0)
        fire_lin(0, jnp.minimum(r0 + (g0 + 2) * 4, rmax))
        wait_sct(1)
        fire_lin(1, jnp.minimum(r0 + (g0 + 3) * 4, rmax))
        return carry

    lax.fori_loop(0, nch // 2, body, 0)
    wait_lin(0)
    wait_lin(1)
    plsc.subcore_barrier()
    st = rows.at[0]
    for k in range(12):
        pltpu.sync_copy(acc.at[pl.ds(zoff + k * 512, 512)], st)
        pltpu.sync_copy(st, out_h.at[c, pl.ds(zoff + k * 512, 512)])
    pltpu.sync_copy(acc.at[pl.ds(zoff + 6144, 256)], rows.at[0, pl.ds(0, 256)])
    pltpu.sync_copy(rows.at[0, pl.ds(0, 256)],
                    out_h.at[c, pl.ds(zoff + 6144, 256)])


_WIDE_SCRATCH = [
    pltpu.VMEM((2, 4, 2, 128), i32),
    pltpu.VMEM((2, 4, 128), f32),
    pltpu.VMEM((2, 512, 16), f32),
    pltpu.VMEM_SHARED((NP, 16), f32),
    pltpu.SemaphoreType.DMA,
    pltpu.SemaphoreType.DMA,
    pltpu.SemaphoreType.DMA,
    pltpu.SemaphoreType.DMA,
    pltpu.SemaphoreType.DMA,
    pltpu.SemaphoreType.DMA,
]


@functools.partial(
    pl.kernel,
    compiler_params=_params,
    out_type=jax.ShapeDtypeStruct((2, NP, 16), f32),
    mesh=_mesh,
    scratch_types=list(_WIDE_SCRATCH),
)
def _prop16_kernel(ei_h, nrm_h, x_h, z2_h, out_h, eibuf, nbuf, rows, acc,
                   s0, s1, s2, s3, s4, s5):
    c = lax.axis_index("c")
    s = lax.axis_index("s")
    wid = c * 16 + s
    r0 = wid * 400

    def fire_gat(p, sem):
        for j in range(4):
            pltpu.async_copy(x_h.at[eibuf.at[p, j, 0]],
                             rows.at[p, pl.ds(j * 128, 128)], sem)

    def wait_gat(p, sem):
        for j in range(4):
            pltpu.make_async_copy(x_h.at[eibuf.at[p, j, 0]],
                                  rows.at[p, pl.ds(j * 128, 128)], sem).wait()

    _prop_wide_body(c, s, ei_h, nrm_h, out_h, z2_h, eibuf, nbuf, rows, acc,
                    (s0, s1, s2, s3, s4, s5), r0, r0 + 400 - 4, 100,
                    fire_gat, wait_gat)


@functools.partial(
    pl.kernel,
    compiler_params=_params,
    out_type=jax.ShapeDtypeStruct((2, NP, 16), f32),
    mesh=_mesh,
    scratch_types=list(_WIDE_SCRATCH),
)
def _prop32_kernel(ei_h, nrm_h, x2_h, z2_h, out_h, eibuf, nbuf, rows, acc,
                   s0, s1, s2, s3, s4, s5):
    c = lax.axis_index("c")
    s = lax.axis_index("s")
    r0 = s * 800

    def fire_gat(p, sem):
        @pl.when(c == 0)
        def _g0():
            for j in range(4):
                pltpu.async_copy(x2_h.at[0].at[eibuf.at[p, j, 0]],
                                 rows.at[p, pl.ds(j * 128, 128)], sem)

        @pl.when(c == 1)
        def _g1():
            for j in range(4):
                pltpu.async_copy(x2_h.at[1].at[eibuf.at[p, j, 0]],
                                 rows.at[p, pl.ds(j * 128, 128)], sem)

    def wait_gat(p, sem):
        for j in range(4):
            pltpu.make_async_copy(x2_h.at[0].at[eibuf.at[p, j, 0]],
                                  rows.at[p, pl.ds(j * 128, 128)], sem).wait()

    _prop_wide_body(c, s, ei_h, nrm_h, out_h, z2_h, eibuf, nbuf, rows, acc,
                    (s0, s1, s2, s3, s4, s5), r0, r0 + 800 - 4, 200,
                    fire_gat, wait_gat)


def _bd(w):
    return jnp.kron(jnp.eye(8, dtype=f32), w)


def _pdense(items, relu, post_w=None):
    o = items[0][1].shape[-1]
    G = o // 16 if o >= 16 else 1
    ow = 128 if o >= 16 else 8 * o
    R = 1600
    nb = P8 // R
    k = len(items)

    wexp = []
    for arr, w in items:
        for g in range(G):
            blk = w[..., 16 * g:16 * (g + 1)] if o >= 16 else w
            if arr.ndim == 3:
                wexp.append(jnp.stack([_bd(blk[t])
                                       for t in range(arr.shape[0])]))
            else:
                wexp.append(_bd(blk))
    if post_w is not None:
        wexp.append(jnp.stack([_bd(post_w[16 * g:16 * (g + 1), :])
                               for g in range(G)]))
    nw = len(wexp)

    def body(*refs):
        ins = refs[:k]
        ws = refs[k:k + nw]
        outs = refs[k + nw:]
        accs = []
        for g in range(G):
            acc = None
            for j, (arr, _) in enumerate(items):
                wr = ws[j * G + g]
                if arr.ndim == 3:
                    t = None
                    for tc in range(arr.shape[0]):
                        d = jnp.dot(ins[j][tc], wr[tc],
                                    preferred_element_type=f32,
                                    precision=lax.Precision.HIGHEST)
                        t = d if t is None else t + d
                else:
                    t = jnp.dot(ins[j][...], wr[...],
                                preferred_element_type=f32,
                                precision=lax.Precision.HIGHEST)
                acc = t if acc is None else acc + t
            if relu:
                acc = jnp.maximum(acc, 0.0)
            accs.append(acc)
        if G > 1:
            for g in range(G):
                outs[0][g, :, :] = accs[g]
        else:
            outs[0][...] = accs[0]
        if post_w is not None:
            pr = ws[k * G]
            pa = None
            for g in range(G):
                d = jnp.dot(accs[g], pr[g], preferred_element_type=f32,
                            precision=lax.Precision.HIGHEST)
                pa = d if pa is None else pa + d
            outs[1][...] = pa

    in_specs = []
    for arr, _ in items:
        if arr.ndim == 3:
            in_specs.append(pl.BlockSpec(
                (arr.shape[0], R, arr.shape[-1]),
                lambda i: (0, i, 0)))
        else:
            in_specs.append(
                pl.BlockSpec((R, arr.shape[-1]), lambda i: (i, 0)))
    for w in wexp:
        in_specs.append(
            pl.BlockSpec(w.shape, lambda i, nd=w.ndim: (0,) * nd))

    out_shapes = []
    out_specs = []
    if G > 1:
        out_shapes.append(jax.ShapeDtypeStruct((G, P8, 128), f32))
        out_specs.append(pl.BlockSpec((G, R, 128), lambda i: (0, i, 0)))
    else:
        out_shapes.append(jax.ShapeDtypeStruct((P8, ow), f32))
        out_specs.append(pl.BlockSpec((R, ow), lambda i: (i, 0)))
    if post_w is not None:
        out_shapes.append(jax.ShapeDtypeStruct((P8, 128), f32))
        out_specs.append(pl.BlockSpec((R, 128), lambda i: (i, 0)))

    res = pl.pallas_call(
        body,
        grid=(nb,),
        in_specs=in_specs,
        out_specs=out_specs,
        out_shape=out_shapes,
    )(*[a for a, _ in items], *wexp)
    return res if len(out_shapes) > 1 else res[0]


def _add2(pk):
    R = 1600
    nb = P8 // R

    def body(p_ref, o_ref):
        o_ref[...] = p_ref[0] + p_ref[1]

    return pl.pallas_call(
        body,
        grid=(nb,),
        in_specs=[pl.BlockSpec((2, R, 128), lambda i: (0, i, 0))],
        out_specs=pl.BlockSpec((R, 128), lambda i: (i, 0)),
        out_shape=jax.ShapeDtypeStruct((P8, 128), f32),
    )(pk)


def kernel(x, edge_index, edge_attr, W1, W2, W3, Wc):
    n = x.shape[0]
    assert n == N and edge_index.shape[1] == E
    row, col = edge_index[0], edge_index[1]
    pad = EP - E
    pidx = (jnp.arange(pad, dtype=i32) * 997) % jnp.int32(n)
    eip = jnp.concatenate(
        [edge_index, jnp.stack([pidx, pidx])], axis=1)
    ei2 = jnp.swapaxes(eip.reshape(2, M, 128), 0, 1)
    ewp = jnp.concatenate([edge_attr, jnp.zeros((pad,), f32)]).reshape(M, 128)

    normp = _norm_kernel(ei2, ewp)

    xv = jnp.pad(x[:, 0], (0, NP - N))
    z1 = jnp.zeros((6400,), f32)
    z2 = jnp.zeros((3200, 16), f32)
    u1 = _prop1_kernel(ei2, normp, xv, z1)
    s1 = _prop1_kernel(ei2, normp, u1, z1)
    h1p = _pdense(
        [(jnp.reshape(xv, (P8, 8)), W1[0] - W1[2]),
         (jnp.reshape(u1, (P8, 8)), W1[1]),
         (jnp.reshape(s1, (P8, 8)), 2.0 * W1[2])],
        relu=True)
    h1 = jnp.reshape(h1p, (NP, 16))

    u2p = _prop16_kernel(ei2, normp, h1, z2)
    u2k = _add2(jnp.reshape(u2p, (2, P8, 128)))
    u2 = jnp.reshape(u2k, (NP, 16))
    s2p = _prop16_kernel(ei2, normp, u2, z2)
    C2 = 2.0 * W2[2]
    h2k = _pdense(
        [(h1p, W2[0] - W2[2]), (u2k, W2[1]),
         (jnp.reshape(s2p, (2, P8, 128)), jnp.stack([C2, C2]))],
        relu=True)
    h2 = jnp.reshape(h2k, (2, NP, 16))

    u3 = _prop32_kernel(ei2, normp, h2, z2)
    s3 = _prop32_kernel(ei2, normp, u3, z2)
    A3 = (W3[0] - W3[2]).reshape(2, 16, 64)
    B3 = W3[1].reshape(2, 16, 64)
    C3 = (2.0 * W3[2]).reshape(2, 16, 64)
    P = jnp.concatenate([Wc[1], Wc[2], jnp.zeros((64, 12), f32)], axis=1)
    h3k, abk = _pdense(
        [(h2k, A3), (jnp.reshape(u3, (2, P8, 128)), B3),
         (jnp.reshape(s3, (2, P8, 128)), C3)],
        relu=True, post_w=P)

    ab = jnp.reshape(abk, (NP, 16))
    qp = _prop16_kernel(ei2, normp, ab, z2)
    qk = _add2(jnp.reshape(qp, (2, P8, 128)))
    q = jnp.reshape(qk, (NP, 16))
    rp = _prop16_kernel(ei2, normp, q, z2)
    D4 = (Wc[0] - Wc[2]).reshape(4, 16, 2)
    S1 = jnp.zeros((16, 2), f32).at[0, 0].set(1.0).at[1, 1].set(1.0)
    S2 = jnp.zeros((16, 2), f32).at[2, 0].set(2.0).at[3, 1].set(2.0)
    outp = _pdense(
        [(h3k, D4), (qk, S1),
         (jnp.reshape(rp, (2, P8, 128)), jnp.stack([S2, S2]))],
        relu=False)
    return jnp.reshape(outp, (NP, 2))[:N]


# speedup vs baseline: 1.5791x; 1.0332x over previous
import functools

import jax
import jax.numpy as jnp
from jax import lax
from jax.experimental import pallas as pl
from jax.experimental.pallas import tpu as pltpu
from jax.experimental.pallas import tpu_sc as plsc

f32 = jnp.float32
i32 = jnp.int32

N = 100000
NP = 102400
P8 = NP // 8
E = 1600000
EP = 1638400
M = EP // 128

_mesh = plsc.VectorSubcoreMesh(core_axis_name="c", subcore_axis_name="s")
_params = pltpu.CompilerParams(needs_layout_passes=False,
                               use_tc_tiling_on_sc=False)


def _nr_rsqrt(x):
    i = lax.bitcast_convert_type(x, i32)
    i = jnp.int32(0x5F3759DF) - lax.shift_right_arithmetic(
        i, jnp.full((16,), 1, i32))
    y = lax.bitcast_convert_type(i, f32)
    for _ in range(3):
        y = y * (1.5 - 0.5 * x * y * y)
    return jnp.where(x > 0.0, y, jnp.zeros_like(y))


@functools.partial(
    pl.kernel,
    compiler_params=_params,
    out_type=jax.ShapeDtypeStruct((M, 128), f32),
    mesh=_mesh,
    scratch_types=[
        pltpu.VMEM((2, 8, 2, 128), i32),
        pltpu.VMEM((2, 8, 128), f32),
        pltpu.VMEM((2, 8, 128), f32),
        pltpu.VMEM((NP,), f32),
        pltpu.VMEM((6400,), f32),
        pltpu.VMEM_SHARED((NP,), f32),
        pltpu.SemaphoreType.DMA,
        pltpu.SemaphoreType.DMA,
        pltpu.SemaphoreType.DMA,
        pltpu.SemaphoreType.DMA,
    ],
)
def _norm_kernel(ei_h, ew_h, out_h, eibuf, ebuf, mbuf, dx, tbuf, dacc,
                 sl0, sl1, sx0, sx1):
    c = lax.axis_index("c")
    s = lax.axis_index("s")
    zoff = s * 6400
    sl = (sl0, sl1)
    sx = (sx0, sx1)

    def fire_lin(p, b):
        pltpu.async_copy(ei_h.at[pl.ds(b, 8)], eibuf.at[p], sl[p])
        pltpu.async_copy(ew_h.at[pl.ds(b, 8)], ebuf.at[p], sl[p])

    def wait_lin(p):
        pltpu.make_async_copy(ei_h.at[pl.ds(0, 8)], eibuf.at[p], sl[p]).wait()
        pltpu.make_async_copy(ew_h.at[pl.ds(0, 8)], ebuf.at[p], sl[p]).wait()

    def _zb(i, carry):
        tbuf[pl.ds(i * 16, 16)] = jnp.zeros((16,), f32)
        return carry

    lax.fori_loop(0, 6400 // 16, _zb, 0)
    pltpu.sync_copy(tbuf, dacc.at[pl.ds(zoff, 6400)])
    plsc.subcore_barrier()

    ra0 = s * 800
    ramax = ra0 + 800 - 8

    def comp_a(p):
        for j in range(8):
            @plsc.parallel_loop(0, 128, 16, unroll=2)
            def _mk(e):
                sv = eibuf[p, j, 0, pl.ds(e, 16)]
                dv = eibuf[p, j, 1, pl.ds(e, 16)]
                ev = ebuf[p, j, pl.ds(e, 16)]
                mbuf[p, j, pl.ds(e, 16)] = jnp.where(sv == dv, 0.0, ev)

    def fire_sct(p):
        for j in range(8):
            pltpu.async_copy(mbuf.at[p, j], dacc.at[eibuf.at[p, j, 0]],
                             sx[p], add=True)

    def wait_sct(p):
        for j in range(8):
            pltpu.make_async_copy(mbuf.at[p, j], dacc.at[eibuf.at[p, j, 0]],
                                  sx[p]).wait()

    fire_lin(0, ra0)
    fire_lin(1, ra0 + 8)

    def body_a(gg, carry):
        g0 = 2 * gg
        wait_lin(0)
        comp_a(0)
        fire_sct(0)
        wait_lin(1)
        comp_a(1)
        fire_sct(1)
        wait_sct(0)
        fire_lin(0, jnp.minimum(ra0 + (g0 + 2) * 8, ramax))
        wait_sct(1)
        fire_lin(1, jnp.minimum(ra0 + (g0 + 3) * 8, ramax))
        return carry

    lax.fori_loop(0, 50, body_a, 0)
    wait_lin(0)
    wait_lin(1)
    plsc.subcore_barrier()

    pltpu.sync_copy(dacc.at[pl.ds(zoff, 6400)], tbuf)
    plsc.subcore_barrier()

    @plsc.parallel_loop(0, 6400, 16, unroll=2)
    def _rs(i):
        tbuf[pl.ds(i, 16)] = _nr_rsqrt(tbuf[pl.ds(i, 16)])

    pltpu.sync_copy(tbuf, dacc.at[pl.ds(zoff, 6400)])
    plsc.subcore_barrier()
    pltpu.sync_copy(dacc, dx)

    rc0 = c * 6400 + s * 400

    def comp_c(p):
        for j in range(8):
            @plsc.parallel_loop(0, 128, 16, unroll=2)
            def _mk(e):
                sv = eibuf[p, j, 0, pl.ds(e, 16)]
                dv = eibuf[p, j, 1, pl.ds(e, 16)]
                ev = ebuf[p, j, pl.ds(e, 16)]
                a = plsc.load_gather(dx, [sv])
                b = plsc.load_gather(dx, [dv])
                ewp = jnp.where(sv == dv, 0.0, ev)
                mbuf[p, j, pl.ds(e, 16)] = (-a) * ewp * b

    def fire_out(p, b):
        pltpu.async_copy(mbuf.at[p], out_h.at[pl.ds(b, 8)], sx[p])

    def wait_out(p):
        pltpu.make_async_copy(mbuf.at[p], out_h.at[pl.ds(0, 8)], sx[p]).wait()

    rcmax = rc0 + 400 - 8
    fire_lin(0, rc0)
    fire_lin(1, rc0 + 8)

    def body_c(gg, carry):
        g0 = 2 * gg
        wait_lin(0)
        comp_c(0)
        fire_out(0, rc0 + g0 * 8)
        wait_lin(1)
        comp_c(1)
        fire_out(1, rc0 + (g0 + 1) * 8)
        wait_out(0)
        fire_lin(0, jnp.minimum(rc0 + (g0 + 2) * 8, rcmax))
        wait_out(1)
        fire_lin(1, jnp.minimum(rc0 + (g0 + 3) * 8, rcmax))
        return carry

    lax.fori_loop(0, 25, body_c, 0)
    wait_lin(0)
    wait_lin(1)


@functools.partial(
    pl.kernel,
    compiler_params=_params,
    out_type=jax.ShapeDtypeStruct((2, NP), f32),
    mesh=_mesh,
    scratch_types=[
        pltpu.VMEM((2, 8, 2, 128), i32),
        pltpu.VMEM((2, 8, 128), f32),
        pltpu.VMEM((2, 8, 128), f32),
        pltpu.VMEM((NP,), f32),
        pltpu.VMEM_SHARED((NP,), f32),
        pltpu.SemaphoreType.DMA,
        pltpu.SemaphoreType.DMA,
        pltpu.SemaphoreType.DMA,
        pltpu.SemaphoreType.DMA,
    ],
)
def _prop1_kernel(ei_h, nrm_h, x_h, z1_h, out_h, eibuf, nbuf, mbuf, xbuf, acc,
                  sl0, sl1, sx0, sx1):
    c = lax.axis_index("c")
    s = lax.axis_index("s")
    zoff = s * 6400
    sl = (sl0, sl1)
    sx = (sx0, sx1)
    pltpu.sync_copy(z1_h, acc.at[pl.ds(zoff, 6400)])
    pltpu.sync_copy(x_h, xbuf)
    plsc.subcore_barrier()

    wid = c * 16 + s
    r0 = wid * 400
    rmax = r0 + 400 - 8

    def fire_lin(p, b):
        pltpu.async_copy(ei_h.at[pl.ds(b, 8)], eibuf.at[p], sl[p])
        pltpu.async_copy(nrm_h.at[pl.ds(b, 8)], nbuf.at[p], sl[p])

    def wait_lin(p):
        pltpu.make_async_copy(ei_h.at[pl.ds(0, 8)], eibuf.at[p], sl[p]).wait()
        pltpu.make_async_copy(nrm_h.at[pl.ds(0, 8)], nbuf.at[p], sl[p]).wait()

    def comp(p):
        for j in range(8):
            @plsc.parallel_loop(0, 128, 16, unroll=2)
            def _mk(e):
                sv = eibuf[p, j, 0, pl.ds(e, 16)]
                xg = plsc.load_gather(xbuf, [sv])
                mbuf[p, j, pl.ds(e, 16)] = xg * nbuf[p, j, pl.ds(e, 16)]

    def fire_sct(p):
        for j in range(8):
            pltpu.async_copy(mbuf.at[p, j], acc.at[eibuf.at[p, j, 1]],
                             sx[p], add=True)

    def wait_sct(p):
        for j in range(8):
            pltpu.make_async_copy(mbuf.at[p, j], acc.at[eibuf.at[p, j, 1]],
                                  sx[p]).wait()

    fire_lin(0, r0)
    fire_lin(1, r0 + 8)

    def body(gg, carry):
        g0 = 2 * gg
        wait_lin(0)
        comp(0)
        fire_sct(0)
        wait_lin(1)
        comp(1)
        fire_sct(1)
        wait_sct(0)
        fire_lin(0, jnp.minimum(r0 + (g0 + 2) * 8, rmax))
        wait_sct(1)
        fire_lin(1, jnp.minimum(r0 + (g0 + 3) * 8, rmax))
        return carry

    lax.fori_loop(0, 25, body, 0)
    wait_lin(0)
    wait_lin(1)
    plsc.subcore_barrier()
    off = s * 6400
    pltpu.sync_copy(acc.at[pl.ds(off, 6400)], xbuf.at[pl.ds(0, 6400)])
    pltpu.sync_copy(xbuf.at[pl.ds(0, 6400)], out_h.at[c, pl.ds(off, 6400)])


def _prop_wide_body(c, s, ei_h, nrm_h, out_h, z2_h, eibuf, nbuf, rows, acc,
                    sems, r0, rmax, nch, fire_gat, wait_gat):
    sl = (sems[0], sems[1])
    sg = (sems[2], sems[3])
    ss = (sems[4], sems[5])
    zoff = s * 6400
    for k in range(2):
        pltpu.sync_copy(z2_h, acc.at[pl.ds(zoff + k * 3200, 3200)])
    plsc.subcore_barrier()

    def fire_lin(p, b):
        pltpu.async_copy(ei_h.at[pl.ds(b, 4)], eibuf.at[p], sl[p])
        pltpu.async_copy(nrm_h.at[pl.ds(b, 4)], nbuf.at[p], sl[p])

    def wait_lin(p):
        pltpu.make_async_copy(ei_h.at[pl.ds(0, 4)], eibuf.at[p], sl[p]).wait()
        pltpu.make_async_copy(nrm_h.at[pl.ds(0, 4)], nbuf.at[p], sl[p]).wait()

    def scale(p):
        for j in range(4):
            @plsc.parallel_loop(0, 128, 16, unroll=2)
            def _sc(e):
                nv = nbuf[p, j, pl.ds(e, 16)]
                for kk in range(16):
                    idx = j * 128 + e + kk
                    rows[p, idx, :] = rows[p, idx, :] * nv[kk]

    def fire_sct(p):
        for j in range(4):
            pltpu.async_copy(rows.at[p, pl.ds(j * 128, 128)],
                             acc.at[eibuf.at[p, j, 1]], ss[p], add=True)

    def wait_sct(p):
        for j in range(4):
            pltpu.make_async_copy(rows.at[p, pl.ds(j * 128, 128)],
                                  acc.at[eibuf.at[p, j, 1]], ss[p]).wait()

    fire_lin(0, r0)
    fire_lin(1, r0 + 4)

    def body(gg, carry):
        g0 = 2 * gg
        wait_lin(0)
        fire_gat(0, sg[0])
        wait_lin(1)
        fire_gat(1, sg[1])
        wait_gat(0, sg[0])
        scale(0)
        fire_sct(0)
        wait_gat(1, sg[1])
        scale(1)
        fire_sct(1)
        wait_sct(0)
        fire_lin(0, jnp.minimum(r0 + (g0 + 2) * 4, rmax))
        wait_sct(1)
        fire_lin(1, jnp.minimum(r0 + (g0 + 3) * 4, rmax))
        return carry

    lax.fori_loop(0, nch // 2, body, 0)
    wait_lin(0)
    wait_lin(1)
    plsc.subcore_barrier()
    st = rows.at[0]
    for k in range(12):
        pltpu.sync_copy(acc.at[pl.ds(zoff + k * 512, 512)], st)
        pltpu.sync_copy(st, out_h.at[c, pl.ds(zoff + k * 512, 512)])
    pltpu.sync_copy(acc.at[pl.ds(zoff + 6144, 256)], rows.at[0, pl.ds(0, 256)])
    pltpu.sync_copy(rows.at[0, pl.ds(0, 256)],
                    out_h.at[c, pl.ds(zoff + 6144, 256)])


_WIDE_SCRATCH = [
    pltpu.VMEM((2, 4, 2, 128), i32),
    pltpu.VMEM((2, 4, 128), f32),
    pltpu.VMEM((2, 512, 16), f32),
    pltpu.VMEM_SHARED((NP, 16), f32),
    pltpu.SemaphoreType.DMA,
    pltpu.SemaphoreType.DMA,
    pltpu.SemaphoreType.DMA,
    pltpu.SemaphoreType.DMA,
    pltpu.SemaphoreType.DMA,
    pltpu.SemaphoreType.DMA,
]


@functools.partial(
    pl.kernel,
    compiler_params=_params,
    out_type=jax.ShapeDtypeStruct((2, NP, 16), f32),
    mesh=_mesh,
    scratch_types=list(_WIDE_SCRATCH),
)
def _prop16_kernel(ei_h, nrm_h, x_h, z2_h, out_h, eibuf, nbuf, rows, acc,
                   s0, s1, s2, s3, s4, s5):
    c = lax.axis_index("c")
    s = lax.axis_index("s")
    wid = c * 16 + s
    r0 = wid * 400

    def fire_gat(p, sem):
        for j in range(4):
            pltpu.async_copy(x_h.at[eibuf.at[p, j, 0]],
                             rows.at[p, pl.ds(j * 128, 128)], sem)

    def wait_gat(p, sem):
        for j in range(4):
            pltpu.make_async_copy(x_h.at[eibuf.at[p, j, 0]],
                                  rows.at[p, pl.ds(j * 128, 128)], sem).wait()

    _prop_wide_body(c, s, ei_h, nrm_h, out_h, z2_h, eibuf, nbuf, rows, acc,
                    (s0, s1, s2, s3, s4, s5), r0, r0 + 400 - 4, 100,
                    fire_gat, wait_gat)


@functools.partial(
    pl.kernel,
    compiler_params=_params,
    out_type=jax.ShapeDtypeStruct((2, NP, 16), f32),
    mesh=_mesh,
    scratch_types=list(_WIDE_SCRATCH),
)
def _prop32_kernel(ei_h, nrm_h, x2_h, z2_h, out_h, eibuf, nbuf, rows, acc,
                   s0, s1, s2, s3, s4, s5):
    c = lax.axis_index("c")
    s = lax.axis_index("s")
    r0 = s * 800

    def fire_gat(p, sem):
        @pl.when(c == 0)
        def _g0():
            for j in range(4):
                pltpu.async_copy(x2_h.at[0].at[eibuf.at[p, j, 0]],
                                 rows.at[p, pl.ds(j * 128, 128)], sem)

        @pl.when(c == 1)
        def _g1():
            for j in range(4):
                pltpu.async_copy(x2_h.at[1].at[eibuf.at[p, j, 0]],
                                 rows.at[p, pl.ds(j * 128, 128)], sem)

    def wait_gat(p, sem):
        for j in range(4):
            pltpu.make_async_copy(x2_h.at[0].at[eibuf.at[p, j, 0]],
                                  rows.at[p, pl.ds(j * 128, 128)], sem).wait()

    _prop_wide_body(c, s, ei_h, nrm_h, out_h, z2_h, eibuf, nbuf, rows, acc,
                    (s0, s1, s2, s3, s4, s5), r0, r0 + 800 - 4, 200,
                    fire_gat, wait_gat)


def _bd(w):
    return jnp.kron(jnp.eye(8, dtype=f32), w)


def _pdense(items, relu, post_w=None):
    o = items[0][1].shape[-1]
    G = o // 16 if o >= 16 else 1
    ow = 128 if o >= 16 else 8 * o
    R = 1600
    nb = P8 // R
    k = len(items)

    wexp = []
    for arr, w in items:
        for g in range(G):
            blk = w[..., 16 * g:16 * (g + 1)] if o >= 16 else w
            if arr.ndim == 3:
                wexp.append(jnp.stack([_bd(blk[t])
                                       for t in range(arr.shape[0])]))
            else:
                wexp.append(_bd(blk))
    if post_w is not None:
        wexp.append(jnp.stack([_bd(post_w[16 * g:16 * (g + 1), :])
                               for g in range(G)]))
    nw = len(wexp)

    def body(*refs):
        ins = refs[:k]
        ws = refs[k:k + nw]
        outs = refs[k + nw:]
        accs = []
        for g in range(G):
            acc = None
            for j, (arr, _) in enumerate(items):
                wr = ws[j * G + g]
                if arr.ndim == 3:
                    t = None
                    for tc in range(arr.shape[0]):
                        d = jnp.dot(ins[j][tc], wr[tc],
                                    preferred_element_type=f32,
                                    precision=lax.Precision.HIGHEST)
                        t = d if t is None else t + d
                else:
                    t = jnp.dot(ins[j][...], wr[...],
                                preferred_element_type=f32,
                                precision=lax.Precision.HIGHEST)
                acc = t if acc is None else acc + t
            if relu:
                acc = jnp.maximum(acc, 0.0)
            accs.append(acc)
        if G > 1:
            for g in range(G):
                outs[0][g, :, :] = accs[g]
        else:
            outs[0][...] = accs[0]
        if post_w is not None:
            pr = ws[k * G]
            pa = None
            for g in range(G):
                d = jnp.dot(accs[g], pr[g], preferred_element_type=f32,
                            precision=lax.Precision.HIGHEST)
                pa = d if pa is None else pa + d
            outs[1][...] = pa

    in_specs = []
    for arr, _ in items:
        if arr.ndim == 3:
            in_specs.append(pl.BlockSpec(
                (arr.shape[0], R, arr.shape[-1]),
                lambda i: (0, i, 0)))
        else:
            in_specs.append(
                pl.BlockSpec((R, arr.shape[-1]), lambda i: (i, 0)))
    for w in wexp:
        in_specs.append(
            pl.BlockSpec(w.shape, lambda i, nd=w.ndim: (0,) * nd))

    out_shapes = []
    out_specs = []
    if G > 1:
        out_shapes.append(jax.ShapeDtypeStruct((G, P8, 128), f32))
        out_specs.append(pl.BlockSpec((G, R, 128), lambda i: (0, i, 0)))
    else:
        out_shapes.append(jax.ShapeDtypeStruct((P8, ow), f32))
        out_specs.append(pl.BlockSpec((R, ow), lambda i: (i, 0)))
    if post_w is not None:
        out_shapes.append(jax.ShapeDtypeStruct((P8, 128), f32))
        out_specs.append(pl.BlockSpec((R, 128), lambda i: (i, 0)))

    res = pl.pallas_call(
        body,
        grid=(nb,),
        in_specs=in_specs,
        out_specs=out_specs,
        out_shape=out_shapes,
    )(*[a for a, _ in items], *wexp)
    return res if len(out_shapes) > 1 else res[0]


def _add2(pk):
    R = 1600
    nb = P8 // R

    def body(p_ref, o_ref):
        o_ref[...] = p_ref[0] + p_ref[1]

    return pl.pallas_call(
        body,
        grid=(nb,),
        in_specs=[pl.BlockSpec((2, R, 128), lambda i: (0, i, 0))],
        out_specs=pl.BlockSpec((R, 128), lambda i: (i, 0)),
        out_shape=jax.ShapeDtypeStruct((P8, 128), f32),
    )(pk)


def _padd1(p2):
    pk = jnp.reshape(p2, (2, NP // 128, 128))

    def body(p_ref, o_ref):
        o_ref[...] = p_ref[0] + p_ref[1]

    out = pl.pallas_call(
        body,
        out_shape=jax.ShapeDtypeStruct((NP // 128, 128), f32),
    )(pk)
    return jnp.reshape(out, (NP,))


def kernel(x, edge_index, edge_attr, W1, W2, W3, Wc):
    n = x.shape[0]
    assert n == N and edge_index.shape[1] == E
    row, col = edge_index[0], edge_index[1]
    pad = EP - E
    pidx = (jnp.arange(pad, dtype=i32) * 997) % jnp.int32(n)
    eip = jnp.concatenate(
        [edge_index, jnp.stack([pidx, pidx])], axis=1)
    ei2 = jnp.swapaxes(eip.reshape(2, M, 128), 0, 1)
    ewp = jnp.concatenate([edge_attr, jnp.zeros((pad,), f32)]).reshape(M, 128)

    normp = _norm_kernel(ei2, ewp)

    xv = jnp.pad(x[:, 0], (0, NP - N))
    z1 = jnp.zeros((6400,), f32)
    z2 = jnp.zeros((3200, 16), f32)
    u1 = _padd1(_prop1_kernel(ei2, normp, xv, z1))
    s1p = _prop1_kernel(ei2, normp, u1, z1)
    C1 = 2.0 * W1[2]
    h1p = _pdense(
        [(jnp.reshape(xv, (P8, 8)), W1[0] - W1[2]),
         (jnp.reshape(u1, (P8, 8)), W1[1]),
         (jnp.reshape(s1p, (2, P8, 8)), jnp.stack([C1, C1]))],
        relu=True)
    h1 = jnp.reshape(h1p, (NP, 16))

    u2p = _prop16_kernel(ei2, normp, h1, z2)
    u2k = _add2(jnp.reshape(u2p, (2, P8, 128)))
    u2 = jnp.reshape(u2k, (NP, 16))
    s2p = _prop16_kernel(ei2, normp, u2, z2)
    C2 = 2.0 * W2[2]
    h2k = _pdense(
        [(h1p, W2[0] - W2[2]), (u2k, W2[1]),
         (jnp.reshape(s2p, (2, P8, 128)), jnp.stack([C2, C2]))],
        relu=True)
    h2 = jnp.reshape(h2k, (2, NP, 16))

    u3 = _prop32_kernel(ei2, normp, h2, z2)
    s3 = _prop32_kernel(ei2, normp, u3, z2)
    A3 = (W3[0] - W3[2]).reshape(2, 16, 64)
    B3 = W3[1].reshape(2, 16, 64)
    C3 = (2.0 * W3[2]).reshape(2, 16, 64)
    P = jnp.concatenate([Wc[1], Wc[2], jnp.zeros((64, 12), f32)], axis=1)
    h3k, abk = _pdense(
        [(h2k, A3), (jnp.reshape(u3, (2, P8, 128)), B3),
         (jnp.reshape(s3, (2, P8, 128)), C3)],
        relu=True, post_w=P)

    ab = jnp.reshape(abk, (NP, 16))
    qp = _prop16_kernel(ei2, normp, ab, z2)
    qk = _add2(jnp.reshape(qp, (2, P8, 128)))
    q = jnp.reshape(qk, (NP, 16))
    rp = _prop16_kernel(ei2, normp, q, z2)
    D4 = (Wc[0] - Wc[2]).reshape(4, 16, 2)
    S1 = jnp.zeros((16, 2), f32).at[0, 0].set(1.0).at[1, 1].set(1.0)
    S2 = jnp.zeros((16, 2), f32).at[2, 0].set(2.0).at[3, 1].set(2.0)
    outp = _pdense(
        [(h3k, D4), (qk, S1),
         (jnp.reshape(rp, (2, P8, 128)), jnp.stack([S2, S2]))],
        relu=False)
    return jnp.reshape(outp, (NP, 2))[:N]


# speedup vs baseline: 1.7983x; 1.1388x over previous
import functools

import jax
import jax.numpy as jnp
from jax import lax
from jax.experimental import pallas as pl
from jax.experimental.pallas import tpu as pltpu
from jax.experimental.pallas import tpu_sc as plsc

f32 = jnp.float32
i32 = jnp.int32

N = 100000
NP = 102400
P8 = NP // 8
E = 1600000
EP = 1638400
M = EP // 128

_mesh = plsc.VectorSubcoreMesh(core_axis_name="c", subcore_axis_name="s")
_params = pltpu.CompilerParams(needs_layout_passes=False,
                               use_tc_tiling_on_sc=False)


def _nr_rsqrt(x):
    i = lax.bitcast_convert_type(x, i32)
    i = jnp.int32(0x5F3759DF) - lax.shift_right_arithmetic(
        i, jnp.full((16,), 1, i32))
    y = lax.bitcast_convert_type(i, f32)
    for _ in range(3):
        y = y * (1.5 - 0.5 * x * y * y)
    return jnp.where(x > 0.0, y, jnp.zeros_like(y))


@functools.partial(
    pl.kernel,
    compiler_params=_params,
    out_type=jax.ShapeDtypeStruct((M, 128), f32),
    mesh=_mesh,
    scratch_types=[
        pltpu.VMEM((2, 8, 2, 128), i32),
        pltpu.VMEM((2, 8, 128), f32),
        pltpu.VMEM((2, 8, 128), f32),
        pltpu.VMEM((NP,), f32),
        pltpu.VMEM((6400,), f32),
        pltpu.VMEM_SHARED((NP,), f32),
        pltpu.SemaphoreType.DMA,
        pltpu.SemaphoreType.DMA,
        pltpu.SemaphoreType.DMA,
        pltpu.SemaphoreType.DMA,
    ],
)
def _norm_kernel(ei_h, ew_h, out_h, eibuf, ebuf, mbuf, dx, tbuf, dacc,
                 sl0, sl1, sx0, sx1):
    c = lax.axis_index("c")
    s = lax.axis_index("s")
    zoff = s * 6400
    sl = (sl0, sl1)
    sx = (sx0, sx1)

    def fire_lin(p, b):
        pltpu.async_copy(ei_h.at[pl.ds(b, 8)], eibuf.at[p], sl[p])
        pltpu.async_copy(ew_h.at[pl.ds(b, 8)], ebuf.at[p], sl[p])

    def wait_lin(p):
        pltpu.make_async_copy(ei_h.at[pl.ds(0, 8)], eibuf.at[p], sl[p]).wait()
        pltpu.make_async_copy(ew_h.at[pl.ds(0, 8)], ebuf.at[p], sl[p]).wait()

    def _zb(i, carry):
        tbuf[pl.ds(i * 16, 16)] = jnp.zeros((16,), f32)
        return carry

    lax.fori_loop(0, 6400 // 16, _zb, 0)
    pltpu.sync_copy(tbuf, dacc.at[pl.ds(zoff, 6400)])
    plsc.subcore_barrier()

    ra0 = s * 800
    ramax = ra0 + 800 - 8

    def comp_a(p):
        for j in range(8):
            @plsc.parallel_loop(0, 128, 16, unroll=2)
            def _mk(e):
                sv = eibuf[p, j, 0, pl.ds(e, 16)]
                dv = eibuf[p, j, 1, pl.ds(e, 16)]
                ev = ebuf[p, j, pl.ds(e, 16)]
                mbuf[p, j, pl.ds(e, 16)] = jnp.where(sv == dv, 0.0, ev)

    def fire_sct(p):
        for j in range(8):
            pltpu.async_copy(mbuf.at[p, j], dacc.at[eibuf.at[p, j, 0]],
                             sx[p], add=True)

    def wait_sct(p):
        for j in range(8):
            pltpu.make_async_copy(mbuf.at[p, j], dacc.at[eibuf.at[p, j, 0]],
                                  sx[p]).wait()

    fire_lin(0, ra0)
    fire_lin(1, ra0 + 8)

    def body_a(gg, carry):
        g0 = 2 * gg
        wait_lin(0)
        comp_a(0)
        fire_sct(0)
        wait_lin(1)
        comp_a(1)
        fire_sct(1)
        wait_sct(0)
        fire_lin(0, jnp.minimum(ra0 + (g0 + 2) * 8, ramax))
        wait_sct(1)
        fire_lin(1, jnp.minimum(ra0 + (g0 + 3) * 8, ramax))
        return carry

    lax.fori_loop(0, 50, body_a, 0)
    wait_lin(0)
    wait_lin(1)
    plsc.subcore_barrier()

    pltpu.sync_copy(dacc.at[pl.ds(zoff, 6400)], tbuf)
    plsc.subcore_barrier()

    @plsc.parallel_loop(0, 6400, 16, unroll=2)
    def _rs(i):
        tbuf[pl.ds(i, 16)] = _nr_rsqrt(tbuf[pl.ds(i, 16)])

    pltpu.sync_copy(tbuf, dacc.at[pl.ds(zoff, 6400)])
    plsc.subcore_barrier()
    pltpu.sync_copy(dacc, dx)

    rc0 = c * 6400 + s * 400

    def comp_c(p):
        for j in range(8):
            @plsc.parallel_loop(0, 128, 16, unroll=2)
            def _mk(e):
                sv = eibuf[p, j, 0, pl.ds(e, 16)]
                dv = eibuf[p, j, 1, pl.ds(e, 16)]
                ev = ebuf[p, j, pl.ds(e, 16)]
                a = plsc.load_gather(dx, [sv])
                b = plsc.load_gather(dx, [dv])
                ewp = jnp.where(sv == dv, 0.0, ev)
                mbuf[p, j, pl.ds(e, 16)] = (-a) * ewp * b

    def fire_out(p, b):
        pltpu.async_copy(mbuf.at[p], out_h.at[pl.ds(b, 8)], sx[p])

    def wait_out(p):
        pltpu.make_async_copy(mbuf.at[p], out_h.at[pl.ds(0, 8)], sx[p]).wait()

    rcmax = rc0 + 400 - 8
    fire_lin(0, rc0)
    fire_lin(1, rc0 + 8)

    def body_c(gg, carry):
        g0 = 2 * gg
        wait_lin(0)
        comp_c(0)
        fire_out(0, rc0 + g0 * 8)
        wait_lin(1)
        comp_c(1)
        fire_out(1, rc0 + (g0 + 1) * 8)
        wait_out(0)
        fire_lin(0, jnp.minimum(rc0 + (g0 + 2) * 8, rcmax))
        wait_out(1)
        fire_lin(1, jnp.minimum(rc0 + (g0 + 3) * 8, rcmax))
        return carry

    lax.fori_loop(0, 25, body_c, 0)
    wait_lin(0)
    wait_lin(1)


@functools.partial(
    pl.kernel,
    compiler_params=_params,
    out_type=jax.ShapeDtypeStruct((2, NP), f32),
    mesh=_mesh,
    scratch_types=[
        pltpu.VMEM((2, 8, 2, 128), i32),
        pltpu.VMEM((2, 8, 128), f32),
        pltpu.VMEM((2, 8, 128), f32),
        pltpu.VMEM((NP,), f32),
        pltpu.VMEM_SHARED((NP,), f32),
        pltpu.SemaphoreType.DMA,
        pltpu.SemaphoreType.DMA,
        pltpu.SemaphoreType.DMA,
        pltpu.SemaphoreType.DMA,
    ],
)
def _prop1_kernel(ei_h, nrm_h, x_h, z1_h, out_h, eibuf, nbuf, mbuf, xbuf, acc,
                  sl0, sl1, sx0, sx1):
    c = lax.axis_index("c")
    s = lax.axis_index("s")
    zoff = s * 6400
    sl = (sl0, sl1)
    sx = (sx0, sx1)
    pltpu.sync_copy(z1_h, acc.at[pl.ds(zoff, 6400)])
    pltpu.sync_copy(x_h, xbuf)
    plsc.subcore_barrier()

    wid = c * 16 + s
    r0 = wid * 400
    rmax = r0 + 400 - 8

    def fire_lin(p, b):
        pltpu.async_copy(ei_h.at[pl.ds(b, 8)], eibuf.at[p], sl[p])
        pltpu.async_copy(nrm_h.at[pl.ds(b, 8)], nbuf.at[p], sl[p])

    def wait_lin(p):
        pltpu.make_async_copy(ei_h.at[pl.ds(0, 8)], eibuf.at[p], sl[p]).wait()
        pltpu.make_async_copy(nrm_h.at[pl.ds(0, 8)], nbuf.at[p], sl[p]).wait()

    def comp(p):
        for j in range(8):
            @plsc.parallel_loop(0, 128, 16, unroll=2)
            def _mk(e):
                sv = eibuf[p, j, 0, pl.ds(e, 16)]
                xg = plsc.load_gather(xbuf, [sv])
                mbuf[p, j, pl.ds(e, 16)] = xg * nbuf[p, j, pl.ds(e, 16)]

    def fire_sct(p):
        for j in range(8):
            pltpu.async_copy(mbuf.at[p, j], acc.at[eibuf.at[p, j, 1]],
                             sx[p], add=True)

    def wait_sct(p):
        for j in range(8):
            pltpu.make_async_copy(mbuf.at[p, j], acc.at[eibuf.at[p, j, 1]],
                                  sx[p]).wait()

    fire_lin(0, r0)
    fire_lin(1, r0 + 8)

    def body(gg, carry):
        g0 = 2 * gg
        wait_lin(0)
        comp(0)
        fire_sct(0)
        wait_lin(1)
        comp(1)
        fire_sct(1)
        wait_sct(0)
        fire_lin(0, jnp.minimum(r0 + (g0 + 2) * 8, rmax))
        wait_sct(1)
        fire_lin(1, jnp.minimum(r0 + (g0 + 3) * 8, rmax))
        return carry

    lax.fori_loop(0, 25, body, 0)
    wait_lin(0)
    wait_lin(1)
    plsc.subcore_barrier()
    off = s * 6400
    pltpu.sync_copy(acc.at[pl.ds(off, 6400)], xbuf.at[pl.ds(0, 6400)])
    pltpu.sync_copy(xbuf.at[pl.ds(0, 6400)], out_h.at[c, pl.ds(off, 6400)])


def _prop_wide_body(c, s, ei_h, nrm_h, out_h, z2_h, sbuf, dbuf, nbuf, rows,
                    acc, sems, r0, rmax, nch, fire_gat, wait_gat):
    ssl = (sems[0], sems[1])
    snl = (sems[2], sems[3])
    sdl = (sems[4], sems[5])
    sg = (sems[6], sems[7])
    ss = (sems[8], sems[9])
    zoff = s * 6400
    for k in range(2):
        pltpu.sync_copy(z2_h, acc.at[pl.ds(zoff + k * 3200, 3200)])
    plsc.subcore_barrier()

    def bb(g):
        return jnp.minimum(r0 + g * 4, rmax)

    def fire_slin(p, b):
        pltpu.async_copy(ei_h.at[pl.ds(b, 4), 0], sbuf.at[p], ssl[p])

    def wait_slin(p):
        pltpu.make_async_copy(ei_h.at[pl.ds(0, 4), 0], sbuf.at[p],
                              ssl[p]).wait()

    def fire_dlin(p, b):
        pltpu.async_copy(ei_h.at[pl.ds(b, 4), 1], dbuf.at[p], sdl[p])

    def wait_dlin(p):
        pltpu.make_async_copy(ei_h.at[pl.ds(0, 4), 1], dbuf.at[p],
                              sdl[p]).wait()

    def fire_nlin(p, b):
        pltpu.async_copy(nrm_h.at[pl.ds(b, 4)], nbuf.at[p], snl[p])

    def wait_nlin(p):
        pltpu.make_async_copy(nrm_h.at[pl.ds(0, 4)], nbuf.at[p],
                              snl[p]).wait()

    def scale(p):
        for j in range(4):
            @plsc.parallel_loop(0, 128, 16, unroll=2)
            def _sc(e):
                nv = nbuf[p, j, pl.ds(e, 16)]
                for kk in range(16):
                    idx = j * 128 + e + kk
                    rows[p, idx, :] = rows[p, idx, :] * nv[kk]

    def fire_sct(p):
        for j in range(4):
            pltpu.async_copy(rows.at[p, pl.ds(j * 128, 128)],
                             acc.at[dbuf.at[p, j]], ss[p], add=True)

    def wait_sct(p):
        for j in range(4):
            pltpu.make_async_copy(rows.at[p, pl.ds(j * 128, 128)],
                                  acc.at[dbuf.at[p, j]], ss[p]).wait()

    for p in range(2):
        fire_slin(p, bb(p))
        fire_nlin(p, bb(p))
        fire_dlin(p, bb(p))
    wait_slin(0)
    fire_gat(0, sg[0])
    wait_slin(1)
    fire_gat(1, sg[1])

    def body(gg, carry):
        g0 = 2 * gg
        wait_gat(0, sg[0])
        fire_slin(0, bb(g0 + 2))
        wait_nlin(0)
        scale(0)
        fire_nlin(0, bb(g0 + 2))
        wait_dlin(0)
        fire_sct(0)
        wait_gat(1, sg[1])
        fire_slin(1, bb(g0 + 3))
        wait_nlin(1)
        scale(1)
        fire_nlin(1, bb(g0 + 3))
        wait_dlin(1)
        fire_sct(1)
        wait_sct(0)
        fire_dlin(0, bb(g0 + 2))
        wait_slin(0)
        fire_gat(0, sg[0])
        wait_sct(1)
        fire_dlin(1, bb(g0 + 3))
        wait_slin(1)
        fire_gat(1, sg[1])
        return carry

    lax.fori_loop(0, nch // 2, body, 0)
    for p in range(2):
        wait_gat(p, sg[p])
        wait_nlin(p)
        wait_dlin(p)
    plsc.subcore_barrier()
    st = rows.at[0]
    for k in range(12):
        pltpu.sync_copy(acc.at[pl.ds(zoff + k * 512, 512)], st)
        pltpu.sync_copy(st, out_h.at[c, pl.ds(zoff + k * 512, 512)])
    pltpu.sync_copy(acc.at[pl.ds(zoff + 6144, 256)], rows.at[0, pl.ds(0, 256)])
    pltpu.sync_copy(rows.at[0, pl.ds(0, 256)],
                    out_h.at[c, pl.ds(zoff + 6144, 256)])


_WIDE_SCRATCH = [
    pltpu.VMEM((2, 4, 128), i32),
    pltpu.VMEM((2, 4, 128), i32),
    pltpu.VMEM((2, 4, 128), f32),
    pltpu.VMEM((2, 512, 16), f32),
    pltpu.VMEM_SHARED((NP, 16), f32),
] + [pltpu.SemaphoreType.DMA] * 10


@functools.partial(
    pl.kernel,
    compiler_params=_params,
    out_type=jax.ShapeDtypeStruct((2, NP, 16), f32),
    mesh=_mesh,
    scratch_types=list(_WIDE_SCRATCH),
)
def _prop16_kernel(ei_h, nrm_h, x_h, z2_h, out_h, sbuf, dbuf, nbuf, rows, acc,
                   *sems):
    c = lax.axis_index("c")
    s = lax.axis_index("s")
    wid = c * 16 + s
    r0 = wid * 400

    def fire_gat(p, sem):
        for j in range(4):
            pltpu.async_copy(x_h.at[sbuf.at[p, j]],
                             rows.at[p, pl.ds(j * 128, 128)], sem)

    def wait_gat(p, sem):
        for j in range(4):
            pltpu.make_async_copy(x_h.at[sbuf.at[p, j]],
                                  rows.at[p, pl.ds(j * 128, 128)], sem).wait()

    _prop_wide_body(c, s, ei_h, nrm_h, out_h, z2_h, sbuf, dbuf, nbuf, rows,
                    acc, sems, r0, r0 + 400 - 4, 100, fire_gat, wait_gat)


@functools.partial(
    pl.kernel,
    compiler_params=_params,
    out_type=jax.ShapeDtypeStruct((2, NP, 16), f32),
    mesh=_mesh,
    scratch_types=list(_WIDE_SCRATCH),
)
def _prop32_kernel(ei_h, nrm_h, x2_h, z2_h, out_h, sbuf, dbuf, nbuf, rows,
                   acc, *sems):
    c = lax.axis_index("c")
    s = lax.axis_index("s")
    r0 = s * 800

    def fire_gat(p, sem):
        @pl.when(c == 0)
        def _g0():
            for j in range(4):
                pltpu.async_copy(x2_h.at[0].at[sbuf.at[p, j]],
                                 rows.at[p, pl.ds(j * 128, 128)], sem)

        @pl.when(c == 1)
        def _g1():
            for j in range(4):
                pltpu.async_copy(x2_h.at[1].at[sbuf.at[p, j]],
                                 rows.at[p, pl.ds(j * 128, 128)], sem)

    def wait_gat(p, sem):
        for j in range(4):
            pltpu.make_async_copy(x2_h.at[0].at[sbuf.at[p, j]],
                                  rows.at[p, pl.ds(j * 128, 128)], sem).wait()

    _prop_wide_body(c, s, ei_h, nrm_h, out_h, z2_h, sbuf, dbuf, nbuf, rows,
                    acc, sems, r0, r0 + 800 - 4, 200, fire_gat, wait_gat)


def _bd(w):
    return jnp.kron(jnp.eye(8, dtype=f32), w)


def _pdense(items, relu, post_w=None):
    o = items[0][1].shape[-1]
    G = o // 16 if o >= 16 else 1
    ow = 128 if o >= 16 else 8 * o
    R = 1600
    nb = P8 // R
    k = len(items)

    wexp = []
    for arr, w in items:
        for g in range(G):
            blk = w[..., 16 * g:16 * (g + 1)] if o >= 16 else w
            if arr.ndim == 3:
                wexp.append(jnp.stack([_bd(blk[t])
                                       for t in range(arr.shape[0])]))
            else:
                wexp.append(_bd(blk))
    if post_w is not None:
        wexp.append(jnp.stack([_bd(post_w[16 * g:16 * (g + 1), :])
                               for g in range(G)]))
    nw = len(wexp)

    def body(*refs):
        ins = refs[:k]
        ws = refs[k:k + nw]
        outs = refs[k + nw:]
        accs = []
        for g in range(G):
            acc = None
            for j, (arr, _) in enumerate(items):
                wr = ws[j * G + g]
                if arr.ndim == 3:
                    t = None
                    for tc in range(arr.shape[0]):
                        d = jnp.dot(ins[j][tc], wr[tc],
                                    preferred_element_type=f32,
                                    precision=lax.Precision.HIGHEST)
                        t = d if t is None else t + d
                else:
                    t = jnp.dot(ins[j][...], wr[...],
                                preferred_element_type=f32,
                                precision=lax.Precision.HIGHEST)
                acc = t if acc is None else acc + t
            if relu:
                acc = jnp.maximum(acc, 0.0)
            accs.append(acc)
        if G > 1:
            for g in range(G):
                outs[0][g, :, :] = accs[g]
        else:
            outs[0][...] = accs[0]
        if post_w is not None:
            pr = ws[k * G]
            pa = None
            for g in range(G):
                d = jnp.dot(accs[g], pr[g], preferred_element_type=f32,
                            precision=lax.Precision.HIGHEST)
                pa = d if pa is None else pa + d
            outs[1][...] = pa

    in_specs = []
    for arr, _ in items:
        if arr.ndim == 3:
            in_specs.append(pl.BlockSpec(
                (arr.shape[0], R, arr.shape[-1]),
                lambda i: (0, i, 0)))
        else:
            in_specs.append(
                pl.BlockSpec((R, arr.shape[-1]), lambda i: (i, 0)))
    for w in wexp:
        in_specs.append(
            pl.BlockSpec(w.shape, lambda i, nd=w.ndim: (0,) * nd))

    out_shapes = []
    out_specs = []
    if G > 1:
        out_shapes.append(jax.ShapeDtypeStruct((G, P8, 128), f32))
        out_specs.append(pl.BlockSpec((G, R, 128), lambda i: (0, i, 0)))
    else:
        out_shapes.append(jax.ShapeDtypeStruct((P8, ow), f32))
        out_specs.append(pl.BlockSpec((R, ow), lambda i: (i, 0)))
    if post_w is not None:
        out_shapes.append(jax.ShapeDtypeStruct((P8, 128), f32))
        out_specs.append(pl.BlockSpec((R, 128), lambda i: (i, 0)))

    res = pl.pallas_call(
        body,
        grid=(nb,),
        in_specs=in_specs,
        out_specs=out_specs,
        out_shape=out_shapes,
    )(*[a for a, _ in items], *wexp)
    return res if len(out_shapes) > 1 else res[0]


def _add2(pk):
    R = 1600
    nb = P8 // R

    def body(p_ref, o_ref):
        o_ref[...] = p_ref[0] + p_ref[1]

    return pl.pallas_call(
        body,
        grid=(nb,),
        in_specs=[pl.BlockSpec((2, R, 128), lambda i: (0, i, 0))],
        out_specs=pl.BlockSpec((R, 128), lambda i: (i, 0)),
        out_shape=jax.ShapeDtypeStruct((P8, 128), f32),
    )(pk)


def _padd1(p2):
    pk = jnp.reshape(p2, (2, NP // 128, 128))

    def body(p_ref, o_ref):
        o_ref[...] = p_ref[0] + p_ref[1]

    out = pl.pallas_call(
        body,
        out_shape=jax.ShapeDtypeStruct((NP // 128, 128), f32),
    )(pk)
    return jnp.reshape(out, (NP,))


def kernel(x, edge_index, edge_attr, W1, W2, W3, Wc):
    n = x.shape[0]
    assert n == N and edge_index.shape[1] == E
    row, col = edge_index[0], edge_index[1]
    pad = EP - E
    pidx = (jnp.arange(pad, dtype=i32) * 997) % jnp.int32(n)
    eip = jnp.concatenate(
        [edge_index, jnp.stack([pidx, pidx])], axis=1)
    ei2 = jnp.swapaxes(eip.reshape(2, M, 128), 0, 1)
    ewp = jnp.concatenate([edge_attr, jnp.zeros((pad,), f32)]).reshape(M, 128)

    normp = _norm_kernel(ei2, ewp)

    xv = jnp.pad(x[:, 0], (0, NP - N))
    z1 = jnp.zeros((6400,), f32)
    z2 = jnp.zeros((3200, 16), f32)
    u1 = _padd1(_prop1_kernel(ei2, normp, xv, z1))
    s1p = _prop1_kernel(ei2, normp, u1, z1)
    C1 = 2.0 * W1[2]
    h1p = _pdense(
        [(jnp.reshape(xv, (P8, 8)), W1[0] - W1[2]),
         (jnp.reshape(u1, (P8, 8)), W1[1]),
         (jnp.reshape(s1p, (2, P8, 8)), jnp.stack([C1, C1]))],
        relu=True)
    h1 = jnp.reshape(h1p, (NP, 16))

    u2p = _prop16_kernel(ei2, normp, h1, z2)
    u2k = _add2(jnp.reshape(u2p, (2, P8, 128)))
    u2 = jnp.reshape(u2k, (NP, 16))
    s2p = _prop16_kernel(ei2, normp, u2, z2)
    C2 = 2.0 * W2[2]
    h2k = _pdense(
        [(h1p, W2[0] - W2[2]), (u2k, W2[1]),
         (jnp.reshape(s2p, (2, P8, 128)), jnp.stack([C2, C2]))],
        relu=True)
    h2 = jnp.reshape(h2k, (2, NP, 16))

    u3 = _prop32_kernel(ei2, normp, h2, z2)
    s3 = _prop32_kernel(ei2, normp, u3, z2)
    A3 = (W3[0] - W3[2]).reshape(2, 16, 64)
    B3 = W3[1].reshape(2, 16, 64)
    C3 = (2.0 * W3[2]).reshape(2, 16, 64)
    P = jnp.concatenate([Wc[1], Wc[2], jnp.zeros((64, 12), f32)], axis=1)
    h3k, abk = _pdense(
        [(h2k, A3), (jnp.reshape(u3, (2, P8, 128)), B3),
         (jnp.reshape(s3, (2, P8, 128)), C3)],
        relu=True, post_w=P)

    ab = jnp.reshape(abk, (NP, 16))
    qp = _prop16_kernel(ei2, normp, ab, z2)
    qk = _add2(jnp.reshape(qp, (2, P8, 128)))
    q = jnp.reshape(qk, (NP, 16))
    rp = _prop16_kernel(ei2, normp, q, z2)
    D4 = (Wc[0] - Wc[2]).reshape(4, 16, 2)
    S1 = jnp.zeros((16, 2), f32).at[0, 0].set(1.0).at[1, 1].set(1.0)
    S2 = jnp.zeros((16, 2), f32).at[2, 0].set(2.0).at[3, 1].set(2.0)
    outp = _pdense(
        [(h3k, D4), (qk, S1),
         (jnp.reshape(rp, (2, P8, 128)), jnp.stack([S2, S2]))],
        relu=False)
    return jnp.reshape(outp, (NP, 2))[:N]
